# Initial kernel scaffold; baseline (speedup 1.0000x reference)
import jax
import jax.numpy as jnp
from jax.experimental import pallas as pl


def kernel(wids, sentences_wids, edge_src, edge_dst, score, emb, Ww, Ws, a_src, a_dst, a_s, a_sup, W_cls, b_cls):
    raise NotImplementedError("write your pallas kernel here")


# speedup vs baseline: 9.8892x; 9.8892x over previous
import functools

import jax
import jax.numpy as jnp
from jax import lax
from jax.experimental import pallas as pl
from jax.experimental.pallas import tpu as pltpu
from jax.experimental.pallas import tpu_sc as plsc

_NC = 2
_NSUB = 16
_LANES = 16
_NT = _NC * _NSUB


def _elu(x):
    return jnp.where(x > 0, x, jnp.exp(x) - 1.0)


def _lrelu(x):
    return jnp.where(x >= 0, x, 0.2 * x)


def _sc_embed(emb, wids, swids_flat, NS, L):
    V, D = emb.shape
    NW = wids.shape[0]
    n_chunks_w = NW // _LANES
    n_rounds_w = (n_chunks_w + _NT - 1) // _NT
    sent_per = NS // _NT
    mesh = plsc.VectorSubcoreMesh(core_axis_name="c", subcore_axis_name="s")

    @functools.partial(
        pl.kernel,
        out_type=[
            jax.ShapeDtypeStruct((NW, D), jnp.float32),
            jax.ShapeDtypeStruct((NS * D,), jnp.float32),
        ],
        mesh=mesh,
        scratch_types=[
            pltpu.VMEM((_LANES,), jnp.int32),
            pltpu.VMEM((_LANES, D), jnp.float32),
            pltpu.VMEM((sent_per * L,), jnp.int32),
            pltpu.VMEM((2 * L, D), jnp.float32),
            pltpu.VMEM((2 * D,), jnp.float32),
        ],
    )
    def body(emb_h, wids_h, swids_h, hw_out, hs_out,
             widx_v, wrows_v, sidx_v, srows_v, hsrow_v):
        c = lax.axis_index("c")
        s = lax.axis_index("s")
        wid = s * _NC + c

        def wbody(t, carry):
            ch = t * _NT + wid

            @pl.when(ch < n_chunks_w)
            def _():
                pltpu.sync_copy(wids_h.at[pl.ds(ch * _LANES, _LANES)], widx_v)
                pltpu.sync_copy(emb_h.at[widx_v], wrows_v)
                pltpu.sync_copy(wrows_v, hw_out.at[pl.ds(ch * _LANES, _LANES)])
            return carry

        lax.fori_loop(0, n_rounds_w, wbody, 0)

        pltpu.sync_copy(swids_h.at[pl.ds(wid * sent_per * L, sent_per * L)],
                        sidx_v)

        def sbody(j, carry):
            pltpu.sync_copy(emb_h.at[sidx_v.at[pl.ds(j * 2 * L, 2 * L)]],
                            srows_v)
            for half in range(2):
                for cc in range(D // _LANES):
                    acc = srows_v[half * L, pl.ds(cc * _LANES, _LANES)]
                    for r in range(1, L):
                        acc = acc + srows_v[half * L + r,
                                            pl.ds(cc * _LANES, _LANES)]
                    hsrow_v[pl.ds(half * D + cc * _LANES, _LANES)] = (
                        acc * (1.0 / L))
            pltpu.sync_copy(
                hsrow_v,
                hs_out.at[pl.ds((wid * sent_per + 2 * j) * D, 2 * D)])
            return carry

        lax.fori_loop(0, sent_per // 2, sbody, 0)

    return body(emb, wids, swids_flat)


def _sc_edges(hw_p, sw, sd, e_src, e_dst, cvec):
    NW, D = hw_p.shape
    NS = sd.shape[0]
    E = e_src.shape[0]
    ept = E // _NT
    nch = ept // _LANES
    rp8 = NS // 8
    mesh = plsc.VectorSubcoreMesh(core_axis_name="c", subcore_axis_name="s")

    @functools.partial(
        pl.kernel,
        out_type=[
            jax.ShapeDtypeStruct((_NC * NS, D), jnp.float32),
            jax.ShapeDtypeStruct((_NC * NS,), jnp.float32),
        ],
        mesh=mesh,
        scratch_types=[
            pltpu.VMEM((NW,), jnp.float32),
            pltpu.VMEM((NS,), jnp.float32),
            pltpu.VMEM((_LANES,), jnp.float32),
            pltpu.VMEM((ept,), jnp.int32),
            pltpu.VMEM((ept,), jnp.int32),
            pltpu.VMEM((2 * _LANES,), jnp.float32),
            pltpu.VMEM((_LANES, D), jnp.float32),
            pltpu.VMEM((rp8, D), jnp.float32),
            pltpu.VMEM((NS,), jnp.float32),
            pltpu.VMEM((_NSUB * NS,), jnp.float32),
            pltpu.VMEM_SHARED((NS, D), jnp.float32),
            pltpu.VMEM_SHARED((_NSUB * NS,), jnp.float32),
        ],
        compiler_params=pltpu.CompilerParams(needs_layout_passes=False),
    )
    def body(hwp_h, sw_h, sd_h, es_h, ed_h, c_h,
             u_out, den_out,
             sw_v, sd_v, c_v, es_v, ed_v, w_v, rows_v,
             stage_v, dloc_v, dall_v, u_sh, dall_sh):
        c = lax.axis_index("c")
        s = lax.axis_index("s")
        wid = s * _NC + c

        zv = jnp.zeros((_LANES,), jnp.float32)

        @pl.when(s < 8)
        def _():
            def zrow(i, carry):
                for cc in range(D // _LANES):
                    stage_v[i, pl.ds(cc * _LANES, _LANES)] = zv
                return carry

            lax.fori_loop(0, rp8, zrow, 0)
            pltpu.sync_copy(stage_v, u_sh.at[pl.ds(s * rp8, rp8)])

        def zden(i, carry):
            dloc_v[pl.ds(i * _LANES, _LANES)] = zv
            return carry

        lax.fori_loop(0, NS // _LANES, zden, 0)

        pltpu.sync_copy(sw_h, sw_v)
        pltpu.sync_copy(sd_h, sd_v)
        pltpu.sync_copy(c_h, c_v)
        pltpu.sync_copy(es_h.at[pl.ds(wid * ept, ept)], es_v)
        pltpu.sync_copy(ed_h.at[pl.ds(wid * ept, ept)], ed_v)
        plsc.subcore_barrier()

        cval = c_v[...]

        def chunk(t, carry):
            sidx = es_v[pl.ds(t * _LANES, _LANES)]
            didx = ed_v[pl.ds(t * _LANES, _LANES)]
            sv = plsc.load_gather(sw_v, [sidx])
            dv = plsc.load_gather(sd_v, [didx])
            x = sv + dv
            e = jnp.where(x >= 0, x, 0.2 * x)
            w = jnp.exp(e - cval)
            w_v[pl.ds(_LANES, _LANES)] = w
            plsc.addupdate_scatter(dloc_v, [didx], w)
            pltpu.sync_copy(hwp_h.at[sidx], rows_v)
            for r in range(_LANES):
                ws = plsc.load_gather(
                    w_v, [jnp.full((_LANES,), _LANES + r, jnp.int32)])
                for cc in range(D // _LANES):
                    sl = pl.ds(cc * _LANES, _LANES)
                    rows_v[r, sl] = rows_v[r, sl] * ws
            pltpu.sync_copy(rows_v, u_sh.at[didx], add=True)
            return carry

        lax.fori_loop(0, nch, chunk, 0)

        pltpu.sync_copy(dloc_v, dall_sh.at[pl.ds(s * NS, NS)])
        plsc.subcore_barrier()

        @pl.when(s < 8)
        def _():
            pltpu.sync_copy(u_sh.at[pl.ds(s * rp8, rp8)], stage_v)
            pltpu.sync_copy(stage_v,
                            u_out.at[pl.ds(c * NS + s * rp8, rp8)])

        @pl.when(s == 8)
        def _():
            pltpu.sync_copy(dall_sh, dall_v)

            def dred(k, carry):
                acc = dall_v[pl.ds(k * _LANES, _LANES)]
                for r in range(1, _NSUB):
                    acc = acc + dall_v[pl.ds(r * NS + k * _LANES, _LANES)]
                dloc_v[pl.ds(k * _LANES, _LANES)] = acc
                return carry

            lax.fori_loop(0, NS // _LANES, dred, 0)
            pltpu.sync_copy(dloc_v, den_out.at[pl.ds(c * NS, NS)])

    return body(hw_p, sw, sd, e_src, e_dst, cvec)


def _tc_word_chain(h_w, Ww, a_src):
    NW, D = h_w.shape
    blk = 800
    grid = NW // blk

    def body(x_ref, w_ref, a_ref, p0_ref, p1_ref, s0_ref, s1_ref, m_ref):
        x = x_ref[...]
        p0 = jnp.dot(x, w_ref[0], preferred_element_type=jnp.float32)
        p0_ref[...] = p0
        s0 = jnp.sum(p0 * a_ref[0:1, :], axis=1, keepdims=True)
        s0_ref[...] = s0
        h1 = _elu(p0)
        p1 = jnp.dot(h1, w_ref[1], preferred_element_type=jnp.float32)
        p1_ref[...] = p1
        s1 = jnp.sum(p1 * a_ref[1:2, :], axis=1, keepdims=True)
        s1_ref[...] = s1
        mx = jnp.concatenate(
            [jnp.max(s0).reshape(1, 1), jnp.max(s1).reshape(1, 1)], axis=1)

        @pl.when(pl.program_id(0) == 0)
        def _():
            m_ref[...] = mx

        @pl.when(pl.program_id(0) > 0)
        def _():
            m_ref[...] = jnp.maximum(m_ref[...], mx)

    return pl.pallas_call(
        body,
        grid=(grid,),
        in_specs=[
            pl.BlockSpec((blk, D), lambda i: (i, 0)),
            pl.BlockSpec((2, D, D), lambda i: (0, 0, 0)),
            pl.BlockSpec((2, D), lambda i: (0, 0)),
        ],
        out_specs=[
            pl.BlockSpec((blk, D), lambda i: (i, 0)),
            pl.BlockSpec((blk, D), lambda i: (i, 0)),
            pl.BlockSpec((blk, 1), lambda i: (i, 0)),
            pl.BlockSpec((blk, 1), lambda i: (i, 0)),
            pl.BlockSpec((1, 2), lambda i: (0, 0)),
        ],
        out_shape=[
            jax.ShapeDtypeStruct((NW, D), jnp.float32),
            jax.ShapeDtypeStruct((NW, D), jnp.float32),
            jax.ShapeDtypeStruct((NW, 1), jnp.float32),
            jax.ShapeDtypeStruct((NW, 1), jnp.float32),
            jax.ShapeDtypeStruct((1, 2), jnp.float32),
        ],
    )(h_w, Ww, a_src)


def _tc_sent_init(h_s, score, Ws0, a_dst0):
    NS, D = h_s.shape
    B, S = score.shape

    def body(hs_ref, sc_ref, w_ref, a_ref, hsp_ref, hsupp_ref, sd_ref, m_ref):
        hs = hs_ref[...]
        hsp = jnp.dot(hs, w_ref[...], preferred_element_type=jnp.float32)
        hsp_ref[...] = hsp
        sd = jnp.sum(hsp * a_ref[...], axis=1, keepdims=True)
        sd_ref[...] = sd
        m_ref[...] = jnp.max(sd).reshape(1, 1)
        rows = []
        for b in range(B):
            sb = sc_ref[b, :].reshape(S, 1)
            rows.append(jnp.sum(sb * hs[b * S:(b + 1) * S, :], axis=0,
                                keepdims=True))
        hsup = jnp.concatenate(rows, axis=0)
        hsupp_ref[...] = jnp.dot(hsup, w_ref[...],
                                 preferred_element_type=jnp.float32)

    return pl.pallas_call(
        body,
        out_shape=[
            jax.ShapeDtypeStruct((NS, D), jnp.float32),
            jax.ShapeDtypeStruct((B, D), jnp.float32),
            jax.ShapeDtypeStruct((NS, 1), jnp.float32),
            jax.ShapeDtypeStruct((1, 1), jnp.float32),
        ],
    )(h_s, score, Ws0, a_dst0)


def _tc_layer0_post(u, den, hs_p, hsup_p, a_s0, a_sup0, Ws1, a_dst1, B, S):
    NS, D = hs_p.shape

    def body(u_ref, d_ref, hsp_ref, hsupp_ref, as_ref, asup_ref,
             w1_ref, ad1_ref, hsp1_ref, hsupp1_ref, sd1_ref, m_ref):
        usum = u_ref[0] + u_ref[1]
        dsum = d_ref[0] + d_ref[1]
        pre = usum / (dsum + 1e-30) + hsp_ref[...]
        hs1 = _elu(pre)
        esup = jnp.sum(hsupp_ref[...] * asup_ref[...], axis=1, keepdims=True)
        ecol = jnp.sum(hs1 * as_ref[...], axis=1, keepdims=True)
        rows = []
        for b in range(B):
            v = ecol[b * S:(b + 1) * S, :] + esup[b, 0]
            lr = _lrelu(v)
            ex = jnp.exp(lr - jnp.max(lr))
            nsb = ex / jnp.sum(ex)
            rows.append(jnp.sum(nsb * hs1[b * S:(b + 1) * S, :], axis=0,
                                keepdims=True))
        hsup1 = _elu(jnp.concatenate(rows, axis=0))
        hsp1 = jnp.dot(hs1, w1_ref[...], preferred_element_type=jnp.float32)
        hsp1_ref[...] = hsp1
        hsupp1_ref[...] = jnp.dot(hsup1, w1_ref[...],
                                  preferred_element_type=jnp.float32)
        sd1 = jnp.sum(hsp1 * ad1_ref[...], axis=1, keepdims=True)
        sd1_ref[...] = sd1
        m_ref[...] = jnp.max(sd1).reshape(1, 1)

    return pl.pallas_call(
        body,
        out_shape=[
            jax.ShapeDtypeStruct((NS, D), jnp.float32),
            jax.ShapeDtypeStruct((B, D), jnp.float32),
            jax.ShapeDtypeStruct((NS, 1), jnp.float32),
            jax.ShapeDtypeStruct((1, 1), jnp.float32),
        ],
    )(u, den, hs_p, hsup_p, a_s0, a_sup0, Ws1, a_dst1)


def _tc_tail(u, den, hs_p, hsup_p, a_s1, a_sup1, W_cls, b_cls, B, S):
    NS, D = hs_p.shape

    def body(u_ref, d_ref, hsp_ref, hsupp_ref, as_ref, asup_ref,
             wc_ref, bc_ref, sh_ref, ns_ref, sup_ref, cls_ref):
        usum = u_ref[0] + u_ref[1]
        dsum = d_ref[0] + d_ref[1]
        pre = usum / (dsum + 1e-30) + hsp_ref[...]
        hs2 = _elu(pre)
        sh_ref[...] = hs2
        esup = jnp.sum(hsupp_ref[...] * asup_ref[...], axis=1, keepdims=True)
        ecol = jnp.sum(hs2 * as_ref[...], axis=1, keepdims=True)
        nt = (((1,), (1,)), ((), ()))
        sup_rows = []
        sps = []
        sq = jnp.zeros((S, S), jnp.float32)
        for b in range(B):
            hb = hs2[b * S:(b + 1) * S, :]
            v = ecol[b * S:(b + 1) * S, :] + esup[b, 0]
            lr = _lrelu(v)
            ex = jnp.exp(lr - jnp.max(lr))
            nsb = ex / jnp.sum(ex)
            ns_ref[b * S:(b + 1) * S, :] = nsb
            sup_rows.append(jnp.sum(nsb * hb, axis=0, keepdims=True))
            wmat = lax.dot_general(nsb, nsb, nt,
                                   preferred_element_type=jnp.float32)
            gram = lax.dot_general(hb, hb, nt,
                                   preferred_element_type=jnp.float32)
            spb = 1.0 / (1.0 + jnp.exp(-(wmat * gram)))
            sps.append(spb)
            sq = sq + spb * spb
        sup_ref[...] = _elu(jnp.concatenate(sup_rows, axis=0))
        nrm = jnp.maximum(jnp.sqrt(sq), 1e-12)
        for b in range(B):
            hb = hs2[b * S:(b + 1) * S, :]
            hagg = jnp.dot(sps[b] / nrm, hb,
                           preferred_element_type=jnp.float32)
            cls_ref[b * S:(b + 1) * S, :] = (
                jnp.dot(hagg, wc_ref[...],
                        preferred_element_type=jnp.float32) + bc_ref[...])

    return pl.pallas_call(
        body,
        out_shape=[
            jax.ShapeDtypeStruct((NS, D), jnp.float32),
            jax.ShapeDtypeStruct((NS, 1), jnp.float32),
            jax.ShapeDtypeStruct((B, D), jnp.float32),
            jax.ShapeDtypeStruct((NS, 2), jnp.float32),
        ],
    )(u, den, hs_p, hsup_p, a_s1, a_sup1, W_cls, b_cls)


_DEBUG_EMU_EDGES = False
_DEBUG_EMU_EMBED = False


def _sc_edges_emu(hw_p, sw, sd, e_src, e_dst, cvec):
    NS = sd.shape[0]
    x = sw[e_src] + sd[e_dst]
    e = jnp.where(x >= 0, x, 0.2 * x)
    w = jnp.exp(e - cvec[0])
    den = jax.ops.segment_sum(w, e_dst, NS)
    u = jax.ops.segment_sum(w[:, None] * hw_p[e_src], e_dst, NS)
    u2 = jnp.concatenate([u, jnp.zeros_like(u)], axis=0)
    den2 = jnp.concatenate([den, jnp.zeros_like(den)], axis=0)
    return u2, den2


def kernel(wids, sentences_wids, edge_src, edge_dst, score, emb,
           Ww, Ws, a_src, a_dst, a_s, a_sup, W_cls, b_cls):
    NW = wids.shape[0]
    NS, L = sentences_wids.shape
    B, S = score.shape
    V, D = emb.shape

    wids = wids.astype(jnp.int32)
    swids = sentences_wids.astype(jnp.int32)
    e_src = edge_src.astype(jnp.int32)
    e_dst = edge_dst.astype(jnp.int32)

    if _DEBUG_EMU_EMBED:
        h_w = jnp.take(emb, wids, axis=0)
        h_s = jnp.mean(jnp.take(emb, swids, axis=0), axis=1)
    else:
        h_w, h_s_flat = _sc_embed(emb, wids, swids.reshape(NS * L), NS, L)
        h_s = h_s_flat.reshape(NS, D)
    edge_fn = _sc_edges_emu if _DEBUG_EMU_EDGES else _sc_edges

    hw_p0, hw_p1, sw0, sw1, msw = _tc_word_chain(h_w, Ww, a_src)
    hs_p0, hsup_p0, sd0, msd0 = _tc_sent_init(h_s, score, Ws[0],
                                              a_dst[0].reshape(1, D))

    c0 = _lrelu(msw[0, 0] + msd0[0, 0])
    u0, den0 = edge_fn(hw_p0, sw0.reshape(NW), sd0.reshape(NS),
                         e_src, e_dst, jnp.broadcast_to(c0, (_LANES,)))

    hs_p1, hsup_p1, sd1, msd1 = _tc_layer0_post(
        u0.reshape(_NC, NS, D),
        den0.reshape(_NC, NS, 1), hs_p0, hsup_p0,
        a_s[0].reshape(1, D), a_sup[0].reshape(1, D),
        Ws[1], a_dst[1].reshape(1, D), B, S)

    c1 = _lrelu(msw[0, 1] + msd1[0, 0])
    u1, den1 = edge_fn(hw_p1, sw1.reshape(NW), sd1.reshape(NS),
                         e_src, e_dst, jnp.broadcast_to(c1, (_LANES,)))

    s_h, ns_col, super_h, cls_out = _tc_tail(
        u1.reshape(_NC, NS, D),
        den1.reshape(_NC, NS, 1), hs_p1, hsup_p1,
        a_s[1].reshape(1, D), a_sup[1].reshape(1, D),
        W_cls, b_cls.reshape(1, 2), B, S)

    new_score = ns_col.reshape(B, S)
    return (new_score, s_h, super_h, cls_out)


# speedup vs baseline: 16.1058x; 1.6286x over previous
import functools

import jax
import jax.numpy as jnp
from jax import lax
from jax.experimental import pallas as pl
from jax.experimental.pallas import tpu as pltpu
from jax.experimental.pallas import tpu_sc as plsc

_NC = 2
_NSUB = 16
_LANES = 16
_NT = _NC * _NSUB


def _elu(x):
    return jnp.where(x > 0, x, jnp.exp(x) - 1.0)


def _lrelu(x):
    return jnp.where(x >= 0, x, 0.2 * x)


def _sc_embed(emb, wids, swids_flat, NS, L):
    V, D = emb.shape
    NW = wids.shape[0]
    n_chunks_w = NW // _LANES
    n_rounds_w = (n_chunks_w + _NT - 1) // _NT
    sent_per = NS // _NT
    mesh = plsc.VectorSubcoreMesh(core_axis_name="c", subcore_axis_name="s")

    @functools.partial(
        pl.kernel,
        out_type=[
            jax.ShapeDtypeStruct((NW, D), jnp.float32),
            jax.ShapeDtypeStruct((NS * D,), jnp.float32),
        ],
        mesh=mesh,
        scratch_types=[
            pltpu.VMEM((_LANES,), jnp.int32),
            pltpu.VMEM((_LANES, D), jnp.float32),
            pltpu.VMEM((sent_per * L,), jnp.int32),
            pltpu.VMEM((2 * L, D), jnp.float32),
            pltpu.VMEM((2 * D,), jnp.float32),
        ],
    )
    def body(emb_h, wids_h, swids_h, hw_out, hs_out,
             widx_v, wrows_v, sidx_v, srows_v, hsrow_v):
        c = lax.axis_index("c")
        s = lax.axis_index("s")
        wid = s * _NC + c

        def wbody(t, carry):
            ch = t * _NT + wid

            @pl.when(ch < n_chunks_w)
            def _():
                pltpu.sync_copy(wids_h.at[pl.ds(ch * _LANES, _LANES)], widx_v)
                pltpu.sync_copy(emb_h.at[widx_v], wrows_v)
                pltpu.sync_copy(wrows_v, hw_out.at[pl.ds(ch * _LANES, _LANES)])
            return carry

        lax.fori_loop(0, n_rounds_w, wbody, 0)

        pltpu.sync_copy(swids_h.at[pl.ds(wid * sent_per * L, sent_per * L)],
                        sidx_v)

        def sbody(j, carry):
            pltpu.sync_copy(emb_h.at[sidx_v.at[pl.ds(j * 2 * L, 2 * L)]],
                            srows_v)
            for half in range(2):
                for cc in range(D // _LANES):
                    acc = srows_v[half * L, pl.ds(cc * _LANES, _LANES)]
                    for r in range(1, L):
                        acc = acc + srows_v[half * L + r,
                                            pl.ds(cc * _LANES, _LANES)]
                    hsrow_v[pl.ds(half * D + cc * _LANES, _LANES)] = (
                        acc * (1.0 / L))
            pltpu.sync_copy(
                hsrow_v,
                hs_out.at[pl.ds((wid * sent_per + 2 * j) * D, 2 * D)])
            return carry

        lax.fori_loop(0, sent_per // 2, sbody, 0)

    return body(emb, wids, swids_flat)


def _sc_edges(hw_p, sw, sd, e_src, e_dst, cvec):
    NW, D = hw_p.shape
    NS = sd.shape[0]
    E = e_src.shape[0]
    ept = E // _NT
    nch = ept // _LANES
    rp8 = NS // 8
    mesh = plsc.VectorSubcoreMesh(core_axis_name="c", subcore_axis_name="s")

    @functools.partial(
        pl.kernel,
        out_type=[
            jax.ShapeDtypeStruct((_NC * NS, D), jnp.float32),
            jax.ShapeDtypeStruct((_NC * NS,), jnp.float32),
        ],
        mesh=mesh,
        scratch_types=[
            pltpu.VMEM((NW,), jnp.float32),
            pltpu.VMEM((NS,), jnp.float32),
            pltpu.VMEM((_LANES,), jnp.float32),
            pltpu.VMEM((ept,), jnp.int32),
            pltpu.VMEM((ept,), jnp.int32),
            pltpu.VMEM((2 * _LANES,), jnp.float32),
            pltpu.VMEM((_LANES, D), jnp.float32),
            pltpu.VMEM((_LANES, D), jnp.float32),
            pltpu.VMEM((_LANES, D), jnp.float32),
            pltpu.VMEM((_LANES, D), jnp.float32),
            pltpu.VMEM((rp8, D), jnp.float32),
            pltpu.VMEM((NS,), jnp.float32),
            pltpu.VMEM((_NSUB * NS,), jnp.float32),
            pltpu.VMEM_SHARED((NS, D), jnp.float32),
            pltpu.VMEM_SHARED((_NSUB * NS,), jnp.float32),
            pltpu.SemaphoreType.DMA,
            pltpu.SemaphoreType.DMA,
            pltpu.SemaphoreType.DMA,
            pltpu.SemaphoreType.DMA,
        ],
        compiler_params=pltpu.CompilerParams(needs_layout_passes=False),
    )
    def body(hwp_h, sw_h, sd_h, es_h, ed_h, c_h,
             u_out, den_out,
             sw_v, sd_v, c_v, es_v, ed_v, w_v,
             gin0_v, gin1_v, sout0_v, sout1_v,
             stage_v, dloc_v, dall_v, u_sh, dall_sh,
             gsem0, gsem1, ssem0, ssem1):
        gin = (gin0_v, gin1_v)
        sout = (sout0_v, sout1_v)
        gsem = (gsem0, gsem1)
        ssem = (ssem0, ssem1)
        c = lax.axis_index("c")
        s = lax.axis_index("s")
        wid = s * _NC + c

        zv = jnp.zeros((_LANES,), jnp.float32)

        @pl.when(s < 8)
        def _():
            def zrow(i, carry):
                for cc in range(D // _LANES):
                    stage_v[i, pl.ds(cc * _LANES, _LANES)] = zv
                return carry

            lax.fori_loop(0, rp8, zrow, 0)
            pltpu.sync_copy(stage_v, u_sh.at[pl.ds(s * rp8, rp8)])

        def zden(i, carry):
            dloc_v[pl.ds(i * _LANES, _LANES)] = zv
            return carry

        lax.fori_loop(0, NS // _LANES, zden, 0)

        pltpu.sync_copy(sw_h, sw_v)
        pltpu.sync_copy(sd_h, sd_v)
        pltpu.sync_copy(c_h, c_v)
        pltpu.sync_copy(es_h.at[pl.ds(wid * ept, ept)], es_v)
        pltpu.sync_copy(ed_h.at[pl.ds(wid * ept, ept)], ed_v)
        plsc.subcore_barrier()

        cval = c_v[...]

        def gather_start(t, b):
            sidx = es_v[pl.ds(t * _LANES, _LANES)]
            pltpu.async_copy(hwp_h.at[sidx], gin[b], gsem[b])

        def gather_wait(t, b):
            sidx = es_v[pl.ds(t * _LANES, _LANES)]
            pltpu.make_async_copy(hwp_h.at[sidx], gin[b], gsem[b]).wait()

        def scatter_start(t, b):
            didx = ed_v[pl.ds(t * _LANES, _LANES)]
            pltpu.async_copy(sout[b], u_sh.at[didx], ssem[b], add=True)

        def scatter_wait(t, b):
            didx = ed_v[pl.ds(t * _LANES, _LANES)]
            pltpu.make_async_copy(sout[b], u_sh.at[didx], ssem[b]).wait()

        gather_start(0, 0)
        gather_start(1, 1)

        def half(t, i, b):
            didx = ed_v[pl.ds(t * _LANES, _LANES)]
            sidx = es_v[pl.ds(t * _LANES, _LANES)]
            sv = plsc.load_gather(sw_v, [sidx])
            dv = plsc.load_gather(sd_v, [didx])
            x = sv + dv
            e = jnp.where(x >= 0, x, 0.2 * x)
            w = jnp.exp(e - cval)
            w_v[pl.ds(_LANES, _LANES)] = w
            plsc.addupdate_scatter(dloc_v, [didx], w)
            gather_wait(t, b)

            @pl.when(i > 0)
            def _():
                scatter_wait(t - 2, b)

            for r in range(_LANES):
                ws = plsc.load_gather(
                    w_v, [jnp.full((_LANES,), _LANES + r, jnp.int32)])
                for cc in range(D // _LANES):
                    sl = pl.ds(cc * _LANES, _LANES)
                    sout[b][r, sl] = gin[b][r, sl] * ws

            @pl.when(t + 2 < nch)
            def _():
                gather_start(t + 2, b)

            scatter_start(t, b)

        def pair(i, carry):
            half(2 * i, i, 0)
            half(2 * i + 1, i, 1)
            return carry

        lax.fori_loop(0, nch // 2, pair, 0)
        scatter_wait(nch - 2, 0)
        scatter_wait(nch - 1, 1)

        pltpu.sync_copy(dloc_v, dall_sh.at[pl.ds(s * NS, NS)])
        plsc.subcore_barrier()

        @pl.when(s < 8)
        def _():
            pltpu.sync_copy(u_sh.at[pl.ds(s * rp8, rp8)], stage_v)
            pltpu.sync_copy(stage_v,
                            u_out.at[pl.ds(c * NS + s * rp8, rp8)])

        @pl.when(s == 8)
        def _():
            pltpu.sync_copy(dall_sh, dall_v)

            def dred(k, carry):
                acc = dall_v[pl.ds(k * _LANES, _LANES)]
                for r in range(1, _NSUB):
                    acc = acc + dall_v[pl.ds(r * NS + k * _LANES, _LANES)]
                dloc_v[pl.ds(k * _LANES, _LANES)] = acc
                return carry

            lax.fori_loop(0, NS // _LANES, dred, 0)
            pltpu.sync_copy(dloc_v, den_out.at[pl.ds(c * NS, NS)])

    return body(hw_p, sw, sd, e_src, e_dst, cvec)


def _tc_word_chain(h_w, Ww, a_src):
    NW, D = h_w.shape
    blk = 800
    grid = NW // blk

    def body(x_ref, w_ref, a_ref, p0_ref, p1_ref, s0_ref, s1_ref, m_ref):
        x = x_ref[...]
        p0 = jnp.dot(x, w_ref[0], preferred_element_type=jnp.float32)
        p0_ref[...] = p0
        s0 = jnp.sum(p0 * a_ref[0:1, :], axis=1, keepdims=True)
        s0_ref[...] = s0
        h1 = _elu(p0)
        p1 = jnp.dot(h1, w_ref[1], preferred_element_type=jnp.float32)
        p1_ref[...] = p1
        s1 = jnp.sum(p1 * a_ref[1:2, :], axis=1, keepdims=True)
        s1_ref[...] = s1
        mx = jnp.concatenate(
            [jnp.max(s0).reshape(1, 1), jnp.max(s1).reshape(1, 1)], axis=1)

        @pl.when(pl.program_id(0) == 0)
        def _():
            m_ref[...] = mx

        @pl.when(pl.program_id(0) > 0)
        def _():
            m_ref[...] = jnp.maximum(m_ref[...], mx)

    return pl.pallas_call(
        body,
        grid=(grid,),
        in_specs=[
            pl.BlockSpec((blk, D), lambda i: (i, 0)),
            pl.BlockSpec((2, D, D), lambda i: (0, 0, 0)),
            pl.BlockSpec((2, D), lambda i: (0, 0)),
        ],
        out_specs=[
            pl.BlockSpec((blk, D), lambda i: (i, 0)),
            pl.BlockSpec((blk, D), lambda i: (i, 0)),
            pl.BlockSpec((blk, 1), lambda i: (i, 0)),
            pl.BlockSpec((blk, 1), lambda i: (i, 0)),
            pl.BlockSpec((1, 2), lambda i: (0, 0)),
        ],
        out_shape=[
            jax.ShapeDtypeStruct((NW, D), jnp.float32),
            jax.ShapeDtypeStruct((NW, D), jnp.float32),
            jax.ShapeDtypeStruct((NW, 1), jnp.float32),
            jax.ShapeDtypeStruct((NW, 1), jnp.float32),
            jax.ShapeDtypeStruct((1, 2), jnp.float32),
        ],
    )(h_w, Ww, a_src)


def _tc_sent_init(h_s, score, Ws0, a_dst0):
    NS, D = h_s.shape
    B, S = score.shape

    def body(hs_ref, sc_ref, w_ref, a_ref, hsp_ref, hsupp_ref, sd_ref, m_ref):
        hs = hs_ref[...]
        hsp = jnp.dot(hs, w_ref[...], preferred_element_type=jnp.float32)
        hsp_ref[...] = hsp
        sd = jnp.sum(hsp * a_ref[...], axis=1, keepdims=True)
        sd_ref[...] = sd
        m_ref[...] = jnp.max(sd).reshape(1, 1)
        rows = []
        for b in range(B):
            sb = sc_ref[b, :].reshape(S, 1)
            rows.append(jnp.sum(sb * hs[b * S:(b + 1) * S, :], axis=0,
                                keepdims=True))
        hsup = jnp.concatenate(rows, axis=0)
        hsupp_ref[...] = jnp.dot(hsup, w_ref[...],
                                 preferred_element_type=jnp.float32)

    return pl.pallas_call(
        body,
        out_shape=[
            jax.ShapeDtypeStruct((NS, D), jnp.float32),
            jax.ShapeDtypeStruct((B, D), jnp.float32),
            jax.ShapeDtypeStruct((NS, 1), jnp.float32),
            jax.ShapeDtypeStruct((1, 1), jnp.float32),
        ],
    )(h_s, score, Ws0, a_dst0)


def _tc_layer0_post(u, den, hs_p, hsup_p, a_s0, a_sup0, Ws1, a_dst1, B, S):
    NS, D = hs_p.shape

    def body(u_ref, d_ref, hsp_ref, hsupp_ref, as_ref, asup_ref,
             w1_ref, ad1_ref, hsp1_ref, hsupp1_ref, sd1_ref, m_ref):
        usum = u_ref[0] + u_ref[1]
        dsum = d_ref[0] + d_ref[1]
        pre = usum / (dsum + 1e-30) + hsp_ref[...]
        hs1 = _elu(pre)
        esup = jnp.sum(hsupp_ref[...] * asup_ref[...], axis=1, keepdims=True)
        ecol = jnp.sum(hs1 * as_ref[...], axis=1, keepdims=True)
        rows = []
        for b in range(B):
            v = ecol[b * S:(b + 1) * S, :] + esup[b, 0]
            lr = _lrelu(v)
            ex = jnp.exp(lr - jnp.max(lr))
            nsb = ex / jnp.sum(ex)
            rows.append(jnp.sum(nsb * hs1[b * S:(b + 1) * S, :], axis=0,
                                keepdims=True))
        hsup1 = _elu(jnp.concatenate(rows, axis=0))
        hsp1 = jnp.dot(hs1, w1_ref[...], preferred_element_type=jnp.float32)
        hsp1_ref[...] = hsp1
        hsupp1_ref[...] = jnp.dot(hsup1, w1_ref[...],
                                  preferred_element_type=jnp.float32)
        sd1 = jnp.sum(hsp1 * ad1_ref[...], axis=1, keepdims=True)
        sd1_ref[...] = sd1
        m_ref[...] = jnp.max(sd1).reshape(1, 1)

    return pl.pallas_call(
        body,
        out_shape=[
            jax.ShapeDtypeStruct((NS, D), jnp.float32),
            jax.ShapeDtypeStruct((B, D), jnp.float32),
            jax.ShapeDtypeStruct((NS, 1), jnp.float32),
            jax.ShapeDtypeStruct((1, 1), jnp.float32),
        ],
    )(u, den, hs_p, hsup_p, a_s0, a_sup0, Ws1, a_dst1)


def _tc_tail(u, den, hs_p, hsup_p, a_s1, a_sup1, W_cls, b_cls, B, S):
    NS, D = hs_p.shape

    def body(u_ref, d_ref, hsp_ref, hsupp_ref, as_ref, asup_ref,
             wc_ref, bc_ref, sh_ref, ns_ref, sup_ref, cls_ref):
        usum = u_ref[0] + u_ref[1]
        dsum = d_ref[0] + d_ref[1]
        pre = usum / (dsum + 1e-30) + hsp_ref[...]
        hs2 = _elu(pre)
        sh_ref[...] = hs2
        esup = jnp.sum(hsupp_ref[...] * asup_ref[...], axis=1, keepdims=True)
        ecol = jnp.sum(hs2 * as_ref[...], axis=1, keepdims=True)
        nt = (((1,), (1,)), ((), ()))
        sup_rows = []
        sps = []
        sq = jnp.zeros((S, S), jnp.float32)
        for b in range(B):
            hb = hs2[b * S:(b + 1) * S, :]
            v = ecol[b * S:(b + 1) * S, :] + esup[b, 0]
            lr = _lrelu(v)
            ex = jnp.exp(lr - jnp.max(lr))
            nsb = ex / jnp.sum(ex)
            ns_ref[b * S:(b + 1) * S, :] = nsb
            sup_rows.append(jnp.sum(nsb * hb, axis=0, keepdims=True))
            wmat = lax.dot_general(nsb, nsb, nt,
                                   preferred_element_type=jnp.float32)
            gram = lax.dot_general(hb, hb, nt,
                                   preferred_element_type=jnp.float32)
            spb = 1.0 / (1.0 + jnp.exp(-(wmat * gram)))
            sps.append(spb)
            sq = sq + spb * spb
        sup_ref[...] = _elu(jnp.concatenate(sup_rows, axis=0))
        nrm = jnp.maximum(jnp.sqrt(sq), 1e-12)
        for b in range(B):
            hb = hs2[b * S:(b + 1) * S, :]
            hagg = jnp.dot(sps[b] / nrm, hb,
                           preferred_element_type=jnp.float32)
            cls_ref[b * S:(b + 1) * S, :] = (
                jnp.dot(hagg, wc_ref[...],
                        preferred_element_type=jnp.float32) + bc_ref[...])

    return pl.pallas_call(
        body,
        out_shape=[
            jax.ShapeDtypeStruct((NS, D), jnp.float32),
            jax.ShapeDtypeStruct((NS, 1), jnp.float32),
            jax.ShapeDtypeStruct((B, D), jnp.float32),
            jax.ShapeDtypeStruct((NS, 2), jnp.float32),
        ],
    )(u, den, hs_p, hsup_p, a_s1, a_sup1, W_cls, b_cls)


_DEBUG_EMU_EDGES = False
_DEBUG_EMU_EMBED = False


def _sc_edges_emu(hw_p, sw, sd, e_src, e_dst, cvec):
    NS = sd.shape[0]
    x = sw[e_src] + sd[e_dst]
    e = jnp.where(x >= 0, x, 0.2 * x)
    w = jnp.exp(e - cvec[0])
    den = jax.ops.segment_sum(w, e_dst, NS)
    u = jax.ops.segment_sum(w[:, None] * hw_p[e_src], e_dst, NS)
    u2 = jnp.concatenate([u, jnp.zeros_like(u)], axis=0)
    den2 = jnp.concatenate([den, jnp.zeros_like(den)], axis=0)
    return u2, den2


def kernel(wids, sentences_wids, edge_src, edge_dst, score, emb,
           Ww, Ws, a_src, a_dst, a_s, a_sup, W_cls, b_cls):
    NW = wids.shape[0]
    NS, L = sentences_wids.shape
    B, S = score.shape
    V, D = emb.shape

    wids = wids.astype(jnp.int32)
    swids = sentences_wids.astype(jnp.int32)
    e_src = edge_src.astype(jnp.int32)
    e_dst = edge_dst.astype(jnp.int32)

    if _DEBUG_EMU_EMBED:
        h_w = jnp.take(emb, wids, axis=0)
        h_s = jnp.mean(jnp.take(emb, swids, axis=0), axis=1)
    else:
        h_w, h_s_flat = _sc_embed(emb, wids, swids.reshape(NS * L), NS, L)
        h_s = h_s_flat.reshape(NS, D)
    edge_fn = _sc_edges_emu if _DEBUG_EMU_EDGES else _sc_edges

    hw_p0, hw_p1, sw0, sw1, msw = _tc_word_chain(h_w, Ww, a_src)
    hs_p0, hsup_p0, sd0, msd0 = _tc_sent_init(h_s, score, Ws[0],
                                              a_dst[0].reshape(1, D))

    c0 = _lrelu(msw[0, 0] + msd0[0, 0])
    u0, den0 = edge_fn(hw_p0, sw0.reshape(NW), sd0.reshape(NS),
                         e_src, e_dst, jnp.broadcast_to(c0, (_LANES,)))

    hs_p1, hsup_p1, sd1, msd1 = _tc_layer0_post(
        u0.reshape(_NC, NS, D),
        den0.reshape(_NC, NS, 1), hs_p0, hsup_p0,
        a_s[0].reshape(1, D), a_sup[0].reshape(1, D),
        Ws[1], a_dst[1].reshape(1, D), B, S)

    c1 = _lrelu(msw[0, 1] + msd1[0, 0])
    u1, den1 = edge_fn(hw_p1, sw1.reshape(NW), sd1.reshape(NS),
                         e_src, e_dst, jnp.broadcast_to(c1, (_LANES,)))

    s_h, ns_col, super_h, cls_out = _tc_tail(
        u1.reshape(_NC, NS, D),
        den1.reshape(_NC, NS, 1), hs_p1, hsup_p1,
        a_s[1].reshape(1, D), a_sup[1].reshape(1, D),
        W_cls, b_cls.reshape(1, 2), B, S)

    new_score = ns_col.reshape(B, S)
    return (new_score, s_h, super_h, cls_out)


# speedup vs baseline: 20.6331x; 1.2811x over previous
import functools

import jax
import jax.numpy as jnp
from jax import lax
from jax.experimental import pallas as pl
from jax.experimental.pallas import tpu as pltpu
from jax.experimental.pallas import tpu_sc as plsc

_NC = 2
_NSUB = 16
_LANES = 16
_NT = _NC * _NSUB


def _elu(x):
    return jnp.where(x > 0, x, jnp.exp(x) - 1.0)


def _lrelu(x):
    return jnp.where(x >= 0, x, 0.2 * x)


def _sc_embed(emb, wids, swids_flat, NS, L):
    V, D = emb.shape
    NW = wids.shape[0]
    n_chunks_w = NW // _LANES
    n_rounds_w = (n_chunks_w + _NT - 1) // _NT
    sent_per = NS // _NT
    mesh = plsc.VectorSubcoreMesh(core_axis_name="c", subcore_axis_name="s")

    @functools.partial(
        pl.kernel,
        out_type=[
            jax.ShapeDtypeStruct((NW, D), jnp.float32),
            jax.ShapeDtypeStruct((NS * D,), jnp.float32),
        ],
        mesh=mesh,
        scratch_types=[
            pltpu.VMEM((_LANES,), jnp.int32),
            pltpu.VMEM((_LANES, D), jnp.float32),
            pltpu.VMEM((sent_per * L,), jnp.int32),
            pltpu.VMEM((2 * L, D), jnp.float32),
            pltpu.VMEM((2 * D,), jnp.float32),
        ],
    )
    def body(emb_h, wids_h, swids_h, hw_out, hs_out,
             widx_v, wrows_v, sidx_v, srows_v, hsrow_v):
        c = lax.axis_index("c")
        s = lax.axis_index("s")
        wid = s * _NC + c

        def wbody(t, carry):
            ch = t * _NT + wid

            @pl.when(ch < n_chunks_w)
            def _():
                pltpu.sync_copy(wids_h.at[pl.ds(ch * _LANES, _LANES)], widx_v)
                pltpu.sync_copy(emb_h.at[widx_v], wrows_v)
                pltpu.sync_copy(wrows_v, hw_out.at[pl.ds(ch * _LANES, _LANES)])
            return carry

        lax.fori_loop(0, n_rounds_w, wbody, 0)

        pltpu.sync_copy(swids_h.at[pl.ds(wid * sent_per * L, sent_per * L)],
                        sidx_v)

        def sbody(j, carry):
            pltpu.sync_copy(emb_h.at[sidx_v.at[pl.ds(j * 2 * L, 2 * L)]],
                            srows_v)
            for half in range(2):
                for cc in range(D // _LANES):
                    acc = srows_v[half * L, pl.ds(cc * _LANES, _LANES)]
                    for r in range(1, L):
                        acc = acc + srows_v[half * L + r,
                                            pl.ds(cc * _LANES, _LANES)]
                    hsrow_v[pl.ds(half * D + cc * _LANES, _LANES)] = (
                        acc * (1.0 / L))
            pltpu.sync_copy(
                hsrow_v,
                hs_out.at[pl.ds((wid * sent_per + 2 * j) * D, 2 * D)])
            return carry

        lax.fori_loop(0, sent_per // 2, sbody, 0)

    return body(emb, wids, swids_flat)


def _sc_edges(hw_p, sw, sd, e_src, e_dst, cvec):
    NW, D = hw_p.shape
    NS = sd.shape[0]
    E = e_src.shape[0]
    ept = E // _NT
    nch = ept // _LANES
    rp8 = NS // 8
    mesh = plsc.VectorSubcoreMesh(core_axis_name="c", subcore_axis_name="s")

    @functools.partial(
        pl.kernel,
        out_type=[
            jax.ShapeDtypeStruct((_NC * NS, D), jnp.float32),
            jax.ShapeDtypeStruct((_NC * NS,), jnp.float32),
        ],
        mesh=mesh,
        scratch_types=[
            pltpu.VMEM((NW,), jnp.float32),
            pltpu.VMEM((NS,), jnp.float32),
            pltpu.VMEM((_LANES,), jnp.float32),
            pltpu.VMEM((ept,), jnp.int32),
            pltpu.VMEM((ept,), jnp.int32),
            pltpu.VMEM((2 * _LANES,), jnp.float32),
            pltpu.VMEM((_LANES, D), jnp.float32),
            pltpu.VMEM((_LANES, D), jnp.float32),
            pltpu.VMEM((_LANES, D), jnp.float32),
            pltpu.VMEM((_LANES, D), jnp.float32),
            pltpu.VMEM((_LANES, D), jnp.float32),
            pltpu.VMEM((_LANES, D), jnp.float32),
            pltpu.VMEM((rp8, D), jnp.float32),
            pltpu.VMEM((NS,), jnp.float32),
            pltpu.VMEM((_NSUB * NS,), jnp.float32),
            pltpu.VMEM_SHARED((NS, D), jnp.float32),
            pltpu.VMEM_SHARED((_NSUB * NS,), jnp.float32),
            pltpu.SemaphoreType.DMA,
            pltpu.SemaphoreType.DMA,
            pltpu.SemaphoreType.DMA,
            pltpu.SemaphoreType.DMA,
            pltpu.SemaphoreType.DMA,
            pltpu.SemaphoreType.DMA,
        ],
        compiler_params=pltpu.CompilerParams(needs_layout_passes=False),
    )
    def body(hwp_h, sw_h, sd_h, es_h, ed_h, c_h,
             u_out, den_out,
             sw_v, sd_v, c_v, es_v, ed_v, w_v,
             gin0_v, gin1_v, gin2_v, gin3_v, sout0_v, sout1_v,
             stage_v, dloc_v, dall_v, u_sh, dall_sh,
             gsem0, gsem1, gsem2, gsem3, ssem0, ssem1):
        gin = (gin0_v, gin1_v, gin2_v, gin3_v)
        sout = (sout0_v, sout1_v)
        gsem = (gsem0, gsem1, gsem2, gsem3)
        ssem = (ssem0, ssem1)
        c = lax.axis_index("c")
        s = lax.axis_index("s")
        wid = s * _NC + c

        zv = jnp.zeros((_LANES,), jnp.float32)

        @pl.when(s < 8)
        def _():
            def zrow(i, carry):
                for cc in range(D // _LANES):
                    stage_v[i, pl.ds(cc * _LANES, _LANES)] = zv
                return carry

            lax.fori_loop(0, rp8, zrow, 0)
            pltpu.sync_copy(stage_v, u_sh.at[pl.ds(s * rp8, rp8)])

        def zden(i, carry):
            dloc_v[pl.ds(i * _LANES, _LANES)] = zv
            return carry

        lax.fori_loop(0, NS // _LANES, zden, 0)

        pltpu.sync_copy(sw_h, sw_v)
        pltpu.sync_copy(sd_h, sd_v)
        pltpu.sync_copy(c_h, c_v)
        pltpu.sync_copy(es_h.at[pl.ds(wid * ept, ept)], es_v)
        pltpu.sync_copy(ed_h.at[pl.ds(wid * ept, ept)], ed_v)
        plsc.subcore_barrier()

        cval = c_v[...]

        def gather_start(t, b):
            sidx = es_v[pl.ds(t * _LANES, _LANES)]
            pltpu.async_copy(hwp_h.at[sidx], gin[b], gsem[b])

        def gather_wait(t, b):
            sidx = es_v[pl.ds(t * _LANES, _LANES)]
            pltpu.make_async_copy(hwp_h.at[sidx], gin[b], gsem[b]).wait()

        def scatter_start(t, b):
            didx = ed_v[pl.ds(t * _LANES, _LANES)]
            pltpu.async_copy(sout[b], u_sh.at[didx], ssem[b], add=True)

        def scatter_wait(t, b):
            didx = ed_v[pl.ds(t * _LANES, _LANES)]
            pltpu.make_async_copy(sout[b], u_sh.at[didx], ssem[b]).wait()

        for k in range(4):
            gather_start(k, k)

        def half(t, i, b, bs, first):
            didx = ed_v[pl.ds(t * _LANES, _LANES)]
            sidx = es_v[pl.ds(t * _LANES, _LANES)]
            sv = plsc.load_gather(sw_v, [sidx])
            dv = plsc.load_gather(sd_v, [didx])
            x = sv + dv
            e = jnp.where(x >= 0, x, 0.2 * x)
            w = jnp.exp(e - cval)
            w_v[pl.ds(_LANES, _LANES)] = w
            plsc.addupdate_scatter(dloc_v, [didx], w)
            gather_wait(t, b)

            if first:
                @pl.when(i > 0)
                def _():
                    scatter_wait(t - 2, bs)
            else:
                scatter_wait(t - 2, bs)

            for r in range(_LANES):
                ws = plsc.load_gather(
                    w_v, [jnp.full((_LANES,), _LANES + r, jnp.int32)])
                for cc in range(D // _LANES):
                    sl = pl.ds(cc * _LANES, _LANES)
                    sout[bs][r, sl] = gin[b][r, sl] * ws

            if isinstance(t, int):
                if t + 4 < nch:
                    gather_start(t + 4, b)
            else:
                @pl.when(t + 4 < nch)
                def _():
                    gather_start(t + 4, b)

            scatter_start(t, bs)

        nq = nch // 4

        def quad(i, carry):
            t0 = 4 * i
            for k in range(4):
                half(t0 + k, i, k, k % 2, k < 2)
            return carry

        lax.fori_loop(0, nq, quad, 0)
        for t in range(4 * nq, nch):
            half(t, 1, t % 4, t % 2, False)
        scatter_wait(nch - 2, (nch - 2) % 2)
        scatter_wait(nch - 1, (nch - 1) % 2)

        pltpu.sync_copy(dloc_v, dall_sh.at[pl.ds(s * NS, NS)])
        plsc.subcore_barrier()

        @pl.when(s < 8)
        def _():
            pltpu.sync_copy(u_sh.at[pl.ds(s * rp8, rp8)], stage_v)
            pltpu.sync_copy(stage_v,
                            u_out.at[pl.ds(c * NS + s * rp8, rp8)])

        @pl.when(s == 8)
        def _():
            pltpu.sync_copy(dall_sh, dall_v)

            def dred(k, carry):
                acc = dall_v[pl.ds(k * _LANES, _LANES)]
                for r in range(1, _NSUB):
                    acc = acc + dall_v[pl.ds(r * NS + k * _LANES, _LANES)]
                dloc_v[pl.ds(k * _LANES, _LANES)] = acc
                return carry

            lax.fori_loop(0, NS // _LANES, dred, 0)
            pltpu.sync_copy(dloc_v, den_out.at[pl.ds(c * NS, NS)])

    return body(hw_p, sw, sd, e_src, e_dst, cvec)


def _tc_word_chain(h_w, Ww, a_src):
    NW, D = h_w.shape
    blk = 800
    grid = NW // blk

    def body(x_ref, w_ref, a_ref, p0_ref, p1_ref, s0_ref, s1_ref, m_ref):
        x = x_ref[...]
        p0 = jnp.dot(x, w_ref[0], preferred_element_type=jnp.float32)
        p0_ref[...] = p0
        s0 = jnp.sum(p0 * a_ref[0:1, :], axis=1, keepdims=True)
        s0_ref[...] = s0
        h1 = _elu(p0)
        p1 = jnp.dot(h1, w_ref[1], preferred_element_type=jnp.float32)
        p1_ref[...] = p1
        s1 = jnp.sum(p1 * a_ref[1:2, :], axis=1, keepdims=True)
        s1_ref[...] = s1
        mx = jnp.concatenate(
            [jnp.max(s0).reshape(1, 1), jnp.max(s1).reshape(1, 1)], axis=1)

        @pl.when(pl.program_id(0) == 0)
        def _():
            m_ref[...] = mx

        @pl.when(pl.program_id(0) > 0)
        def _():
            m_ref[...] = jnp.maximum(m_ref[...], mx)

    return pl.pallas_call(
        body,
        grid=(grid,),
        in_specs=[
            pl.BlockSpec((blk, D), lambda i: (i, 0)),
            pl.BlockSpec((2, D, D), lambda i: (0, 0, 0)),
            pl.BlockSpec((2, D), lambda i: (0, 0)),
        ],
        out_specs=[
            pl.BlockSpec((blk, D), lambda i: (i, 0)),
            pl.BlockSpec((blk, D), lambda i: (i, 0)),
            pl.BlockSpec((blk, 1), lambda i: (i, 0)),
            pl.BlockSpec((blk, 1), lambda i: (i, 0)),
            pl.BlockSpec((1, 2), lambda i: (0, 0)),
        ],
        out_shape=[
            jax.ShapeDtypeStruct((NW, D), jnp.float32),
            jax.ShapeDtypeStruct((NW, D), jnp.float32),
            jax.ShapeDtypeStruct((NW, 1), jnp.float32),
            jax.ShapeDtypeStruct((NW, 1), jnp.float32),
            jax.ShapeDtypeStruct((1, 2), jnp.float32),
        ],
    )(h_w, Ww, a_src)


def _tc_sent_init(h_s, score, Ws0, a_dst0):
    NS, D = h_s.shape
    B, S = score.shape

    def body(hs_ref, sc_ref, w_ref, a_ref, hsp_ref, hsupp_ref, sd_ref, m_ref):
        hs = hs_ref[...]
        hsp = jnp.dot(hs, w_ref[...], preferred_element_type=jnp.float32)
        hsp_ref[...] = hsp
        sd = jnp.sum(hsp * a_ref[...], axis=1, keepdims=True)
        sd_ref[...] = sd
        m_ref[...] = jnp.max(sd).reshape(1, 1)
        rows = []
        for b in range(B):
            sb = sc_ref[b, :].reshape(S, 1)
            rows.append(jnp.sum(sb * hs[b * S:(b + 1) * S, :], axis=0,
                                keepdims=True))
        hsup = jnp.concatenate(rows, axis=0)
        hsupp_ref[...] = jnp.dot(hsup, w_ref[...],
                                 preferred_element_type=jnp.float32)

    return pl.pallas_call(
        body,
        out_shape=[
            jax.ShapeDtypeStruct((NS, D), jnp.float32),
            jax.ShapeDtypeStruct((B, D), jnp.float32),
            jax.ShapeDtypeStruct((NS, 1), jnp.float32),
            jax.ShapeDtypeStruct((1, 1), jnp.float32),
        ],
    )(h_s, score, Ws0, a_dst0)


def _tc_layer0_post(u, den, hs_p, hsup_p, a_s0, a_sup0, Ws1, a_dst1, B, S):
    NS, D = hs_p.shape

    def body(u_ref, d_ref, hsp_ref, hsupp_ref, as_ref, asup_ref,
             w1_ref, ad1_ref, hsp1_ref, hsupp1_ref, sd1_ref, m_ref):
        usum = u_ref[0] + u_ref[1]
        dsum = d_ref[0] + d_ref[1]
        pre = usum / (dsum + 1e-30) + hsp_ref[...]
        hs1 = _elu(pre)
        esup = jnp.sum(hsupp_ref[...] * asup_ref[...], axis=1, keepdims=True)
        ecol = jnp.sum(hs1 * as_ref[...], axis=1, keepdims=True)
        rows = []
        for b in range(B):
            v = ecol[b * S:(b + 1) * S, :] + esup[b, 0]
            lr = _lrelu(v)
            ex = jnp.exp(lr - jnp.max(lr))
            nsb = ex / jnp.sum(ex)
            rows.append(jnp.sum(nsb * hs1[b * S:(b + 1) * S, :], axis=0,
                                keepdims=True))
        hsup1 = _elu(jnp.concatenate(rows, axis=0))
        hsp1 = jnp.dot(hs1, w1_ref[...], preferred_element_type=jnp.float32)
        hsp1_ref[...] = hsp1
        hsupp1_ref[...] = jnp.dot(hsup1, w1_ref[...],
                                  preferred_element_type=jnp.float32)
        sd1 = jnp.sum(hsp1 * ad1_ref[...], axis=1, keepdims=True)
        sd1_ref[...] = sd1
        m_ref[...] = jnp.max(sd1).reshape(1, 1)

    return pl.pallas_call(
        body,
        out_shape=[
            jax.ShapeDtypeStruct((NS, D), jnp.float32),
            jax.ShapeDtypeStruct((B, D), jnp.float32),
            jax.ShapeDtypeStruct((NS, 1), jnp.float32),
            jax.ShapeDtypeStruct((1, 1), jnp.float32),
        ],
    )(u, den, hs_p, hsup_p, a_s0, a_sup0, Ws1, a_dst1)


def _tc_tail(u, den, hs_p, hsup_p, a_s1, a_sup1, W_cls, b_cls, B, S):
    NS, D = hs_p.shape

    def body(u_ref, d_ref, hsp_ref, hsupp_ref, as_ref, asup_ref,
             wc_ref, bc_ref, sh_ref, ns_ref, sup_ref, cls_ref):
        usum = u_ref[0] + u_ref[1]
        dsum = d_ref[0] + d_ref[1]
        pre = usum / (dsum + 1e-30) + hsp_ref[...]
        hs2 = _elu(pre)
        sh_ref[...] = hs2
        esup = jnp.sum(hsupp_ref[...] * asup_ref[...], axis=1, keepdims=True)
        ecol = jnp.sum(hs2 * as_ref[...], axis=1, keepdims=True)
        nt = (((1,), (1,)), ((), ()))
        sup_rows = []
        sps = []
        sq = jnp.zeros((S, S), jnp.float32)
        for b in range(B):
            hb = hs2[b * S:(b + 1) * S, :]
            v = ecol[b * S:(b + 1) * S, :] + esup[b, 0]
            lr = _lrelu(v)
            ex = jnp.exp(lr - jnp.max(lr))
            nsb = ex / jnp.sum(ex)
            ns_ref[b * S:(b + 1) * S, :] = nsb
            sup_rows.append(jnp.sum(nsb * hb, axis=0, keepdims=True))
            wmat = lax.dot_general(nsb, nsb, nt,
                                   preferred_element_type=jnp.float32)
            gram = lax.dot_general(hb, hb, nt,
                                   preferred_element_type=jnp.float32)
            spb = 1.0 / (1.0 + jnp.exp(-(wmat * gram)))
            sps.append(spb)
            sq = sq + spb * spb
        sup_ref[...] = _elu(jnp.concatenate(sup_rows, axis=0))
        nrm = jnp.maximum(jnp.sqrt(sq), 1e-12)
        for b in range(B):
            hb = hs2[b * S:(b + 1) * S, :]
            hagg = jnp.dot(sps[b] / nrm, hb,
                           preferred_element_type=jnp.float32)
            cls_ref[b * S:(b + 1) * S, :] = (
                jnp.dot(hagg, wc_ref[...],
                        preferred_element_type=jnp.float32) + bc_ref[...])

    return pl.pallas_call(
        body,
        out_shape=[
            jax.ShapeDtypeStruct((NS, D), jnp.float32),
            jax.ShapeDtypeStruct((NS, 1), jnp.float32),
            jax.ShapeDtypeStruct((B, D), jnp.float32),
            jax.ShapeDtypeStruct((NS, 2), jnp.float32),
        ],
    )(u, den, hs_p, hsup_p, a_s1, a_sup1, W_cls, b_cls)


_DEBUG_EMU_EDGES = False
_DEBUG_EMU_EMBED = False


def _sc_edges_emu(hw_p, sw, sd, e_src, e_dst, cvec):
    NS = sd.shape[0]
    x = sw[e_src] + sd[e_dst]
    e = jnp.where(x >= 0, x, 0.2 * x)
    w = jnp.exp(e - cvec[0])
    den = jax.ops.segment_sum(w, e_dst, NS)
    u = jax.ops.segment_sum(w[:, None] * hw_p[e_src], e_dst, NS)
    u2 = jnp.concatenate([u, jnp.zeros_like(u)], axis=0)
    den2 = jnp.concatenate([den, jnp.zeros_like(den)], axis=0)
    return u2, den2


def kernel(wids, sentences_wids, edge_src, edge_dst, score, emb,
           Ww, Ws, a_src, a_dst, a_s, a_sup, W_cls, b_cls):
    NW = wids.shape[0]
    NS, L = sentences_wids.shape
    B, S = score.shape
    V, D = emb.shape

    wids = wids.astype(jnp.int32)
    swids = sentences_wids.astype(jnp.int32)
    e_src = edge_src.astype(jnp.int32)
    e_dst = edge_dst.astype(jnp.int32)

    if _DEBUG_EMU_EMBED:
        h_w = jnp.take(emb, wids, axis=0)
        h_s = jnp.mean(jnp.take(emb, swids, axis=0), axis=1)
    else:
        h_w, h_s_flat = _sc_embed(emb, wids, swids.reshape(NS * L), NS, L)
        h_s = h_s_flat.reshape(NS, D)
    edge_fn = _sc_edges_emu if _DEBUG_EMU_EDGES else _sc_edges

    hw_p0, hw_p1, sw0, sw1, msw = _tc_word_chain(h_w, Ww, a_src)
    hs_p0, hsup_p0, sd0, msd0 = _tc_sent_init(h_s, score, Ws[0],
                                              a_dst[0].reshape(1, D))

    c0 = _lrelu(msw[0, 0] + msd0[0, 0])
    u0, den0 = edge_fn(hw_p0, sw0.reshape(NW), sd0.reshape(NS),
                         e_src, e_dst, jnp.broadcast_to(c0, (_LANES,)))

    hs_p1, hsup_p1, sd1, msd1 = _tc_layer0_post(
        u0.reshape(_NC, NS, D),
        den0.reshape(_NC, NS, 1), hs_p0, hsup_p0,
        a_s[0].reshape(1, D), a_sup[0].reshape(1, D),
        Ws[1], a_dst[1].reshape(1, D), B, S)

    c1 = _lrelu(msw[0, 1] + msd1[0, 0])
    u1, den1 = edge_fn(hw_p1, sw1.reshape(NW), sd1.reshape(NS),
                         e_src, e_dst, jnp.broadcast_to(c1, (_LANES,)))

    s_h, ns_col, super_h, cls_out = _tc_tail(
        u1.reshape(_NC, NS, D),
        den1.reshape(_NC, NS, 1), hs_p1, hsup_p1,
        a_s[1].reshape(1, D), a_sup[1].reshape(1, D),
        W_cls, b_cls.reshape(1, 2), B, S)

    new_score = ns_col.reshape(B, S)
    return (new_score, s_h, super_h, cls_out)


# speedup vs baseline: 23.5689x; 1.1423x over previous
import functools

import jax
import jax.numpy as jnp
from jax import lax
from jax.experimental import pallas as pl
from jax.experimental.pallas import tpu as pltpu
from jax.experimental.pallas import tpu_sc as plsc

_NC = 2
_NSUB = 16
_LANES = 16
_NT = _NC * _NSUB


def _elu(x):
    return jnp.where(x > 0, x, jnp.exp(x) - 1.0)


def _lrelu(x):
    return jnp.where(x >= 0, x, 0.2 * x)


def _sc_embed_words(emb, wids):
    V, D = emb.shape
    NW = wids.shape[0]
    wpt = 640
    gsz = 64
    mesh = plsc.VectorSubcoreMesh(core_axis_name="c", subcore_axis_name="s")

    @functools.partial(
        pl.kernel,
        out_type=jax.ShapeDtypeStruct((NW, D), jnp.float32),
        mesh=mesh,
        scratch_types=[
            pltpu.VMEM((wpt,), jnp.int32),
            pltpu.VMEM((gsz, D), jnp.float32),
            pltpu.VMEM((gsz, D), jnp.float32),
            pltpu.SemaphoreType.DMA,
            pltpu.SemaphoreType.DMA,
            pltpu.SemaphoreType.DMA,
            pltpu.SemaphoreType.DMA,
        ],
    )
    def body(emb_h, wids_h, hw_out, widx_v, r0_v, r1_v, g0, g1, w0, w1):
        c = lax.axis_index("c")
        s = lax.axis_index("s")
        wid = s * _NC + c
        base = wid * wpt
        n_ch = wpt // gsz
        rows = (r0_v, r1_v)
        gsem = (g0, g1)
        wsem = (w0, w1)

        pltpu.sync_copy(wids_h.at[pl.ds(base, wpt)], widx_v)

        def g_start(k, b):
            pltpu.async_copy(emb_h.at[widx_v.at[pl.ds(k * gsz, gsz)]],
                             rows[b], gsem[b])

        def g_wait(k, b):
            pltpu.make_async_copy(emb_h.at[widx_v.at[pl.ds(k * gsz, gsz)]],
                                  rows[b], gsem[b]).wait()

        def w_start(k, b):
            pltpu.async_copy(rows[b], hw_out.at[pl.ds(base + k * gsz, gsz)],
                             wsem[b])

        def w_wait(k, b):
            pltpu.make_async_copy(rows[b],
                                  hw_out.at[pl.ds(base + k * gsz, gsz)],
                                  wsem[b]).wait()

        g_start(0, 0)
        g_start(1, 1)

        def step(k, carry):
            for b in range(2):
                t = 2 * k + b
                g_wait(t, b)
                w_start(t, b)

                @pl.when(t + 2 < n_ch)
                def _():
                    w_wait(t, b)
                    g_start(t + 2, b)
            return carry

        lax.fori_loop(0, n_ch // 2, step, 0)
        w_wait(n_ch - 2, 0)
        w_wait(n_ch - 1, 1)

    return body(emb, wids)


def _sc_embed_sents(emb, swids_flat, NS, L):
    V, D = emb.shape
    sent_per = NS // _NT
    nst = sent_per // 2
    mesh = plsc.VectorSubcoreMesh(core_axis_name="c", subcore_axis_name="s")

    @functools.partial(
        pl.kernel,
        out_type=jax.ShapeDtypeStruct((NS * D,), jnp.float32),
        mesh=mesh,
        scratch_types=[
            pltpu.VMEM((sent_per * L,), jnp.int32),
            pltpu.VMEM((2 * L, D), jnp.float32),
            pltpu.VMEM((2 * L, D), jnp.float32),
            pltpu.VMEM((sent_per * D,), jnp.float32),
            pltpu.SemaphoreType.DMA,
            pltpu.SemaphoreType.DMA,
        ],
    )
    def body(emb_h, swids_h, hs_out, sidx_v, s0_v, s1_v, pool_v, g0, g1):
        c = lax.axis_index("c")
        s = lax.axis_index("s")
        wid = s * _NC + c
        srows = (s0_v, s1_v)
        gsem = (g0, g1)

        pltpu.sync_copy(swids_h.at[pl.ds(wid * sent_per * L, sent_per * L)],
                        sidx_v)

        def g_start(j, b):
            pltpu.async_copy(emb_h.at[sidx_v.at[pl.ds(j * 2 * L, 2 * L)]],
                             srows[b], gsem[b])

        def g_wait(j, b):
            pltpu.make_async_copy(
                emb_h.at[sidx_v.at[pl.ds(j * 2 * L, 2 * L)]],
                srows[b], gsem[b]).wait()

        g_start(0, 0)
        g_start(1, 1)

        def pool2(j, b):
            g_wait(j, b)
            for half in range(2):
                for cc in range(D // _LANES):
                    acc = srows[b][half * L, pl.ds(cc * _LANES, _LANES)]
                    for r in range(1, L):
                        acc = acc + srows[b][half * L + r,
                                             pl.ds(cc * _LANES, _LANES)]
                    pool_v[pl.ds((2 * j + half) * D + cc * _LANES,
                                 _LANES)] = acc * (1.0 / L)

            if isinstance(j, int):
                if j + 2 < nst:
                    g_start(j + 2, b)
            else:
                @pl.when(j + 2 < nst)
                def _():
                    g_start(j + 2, b)

        def step(k, carry):
            pool2(2 * k, 0)
            pool2(2 * k + 1, 1)
            return carry

        lax.fori_loop(0, nst // 2, step, 0)
        for j in range(2 * (nst // 2), nst):
            pool2(j, j % 2)
        pltpu.sync_copy(pool_v,
                        hs_out.at[pl.ds(wid * sent_per * D, sent_per * D)])

    return body(emb, swids_flat)


def _sc_edges(hw_p, sw, sd, e_src, e_dst, cvec):
    NW, D = hw_p.shape
    NS = sd.shape[0]
    E = e_src.shape[0]
    ept = E // _NT
    nch = ept // _LANES
    rp8 = NS // 8
    mesh = plsc.VectorSubcoreMesh(core_axis_name="c", subcore_axis_name="s")

    @functools.partial(
        pl.kernel,
        out_type=[
            jax.ShapeDtypeStruct((_NC * NS, D), jnp.float32),
            jax.ShapeDtypeStruct((_NC * NS,), jnp.float32),
        ],
        mesh=mesh,
        scratch_types=[
            pltpu.VMEM((NW,), jnp.float32),
            pltpu.VMEM((NS,), jnp.float32),
            pltpu.VMEM((_LANES,), jnp.float32),
            pltpu.VMEM((ept,), jnp.int32),
            pltpu.VMEM((ept,), jnp.int32),
            pltpu.VMEM((2 * _LANES,), jnp.float32),
            pltpu.VMEM((_LANES, D), jnp.float32),
            pltpu.VMEM((_LANES, D), jnp.float32),
            pltpu.VMEM((_LANES, D), jnp.float32),
            pltpu.VMEM((_LANES, D), jnp.float32),
            pltpu.VMEM((_LANES, D), jnp.float32),
            pltpu.VMEM((_LANES, D), jnp.float32),
            pltpu.VMEM((rp8, D), jnp.float32),
            pltpu.VMEM((NS,), jnp.float32),
            pltpu.VMEM((_NSUB * NS,), jnp.float32),
            pltpu.VMEM_SHARED((NS, D), jnp.float32),
            pltpu.VMEM_SHARED((_NSUB * NS,), jnp.float32),
            pltpu.SemaphoreType.DMA,
            pltpu.SemaphoreType.DMA,
            pltpu.SemaphoreType.DMA,
            pltpu.SemaphoreType.DMA,
            pltpu.SemaphoreType.DMA,
            pltpu.SemaphoreType.DMA,
        ],
        compiler_params=pltpu.CompilerParams(needs_layout_passes=False),
    )
    def body(hwp_h, sw_h, sd_h, es_h, ed_h, c_h,
             u_out, den_out,
             sw_v, sd_v, c_v, es_v, ed_v, w_v,
             gin0_v, gin1_v, gin2_v, gin3_v, sout0_v, sout1_v,
             stage_v, dloc_v, dall_v, u_sh, dall_sh,
             gsem0, gsem1, gsem2, gsem3, ssem0, ssem1):
        gin = (gin0_v, gin1_v, gin2_v, gin3_v)
        sout = (sout0_v, sout1_v)
        gsem = (gsem0, gsem1, gsem2, gsem3)
        ssem = (ssem0, ssem1)
        c = lax.axis_index("c")
        s = lax.axis_index("s")
        wid = s * _NC + c

        zv = jnp.zeros((_LANES,), jnp.float32)

        @pl.when(s < 8)
        def _():
            def zrow(i, carry):
                for cc in range(D // _LANES):
                    stage_v[i, pl.ds(cc * _LANES, _LANES)] = zv
                return carry

            lax.fori_loop(0, rp8, zrow, 0)
            pltpu.sync_copy(stage_v, u_sh.at[pl.ds(s * rp8, rp8)])

        def zden(i, carry):
            dloc_v[pl.ds(i * _LANES, _LANES)] = zv
            return carry

        lax.fori_loop(0, NS // _LANES, zden, 0)

        pltpu.sync_copy(sw_h, sw_v)
        pltpu.sync_copy(sd_h, sd_v)
        pltpu.sync_copy(c_h, c_v)
        pltpu.sync_copy(es_h.at[pl.ds(wid * ept, ept)], es_v)
        pltpu.sync_copy(ed_h.at[pl.ds(wid * ept, ept)], ed_v)
        plsc.subcore_barrier()

        cval = c_v[...]

        def gather_start(t, b):
            sidx = es_v[pl.ds(t * _LANES, _LANES)]
            pltpu.async_copy(hwp_h.at[sidx], gin[b], gsem[b])

        def gather_wait(t, b):
            sidx = es_v[pl.ds(t * _LANES, _LANES)]
            pltpu.make_async_copy(hwp_h.at[sidx], gin[b], gsem[b]).wait()

        def scatter_start(t, b):
            didx = ed_v[pl.ds(t * _LANES, _LANES)]
            pltpu.async_copy(sout[b], u_sh.at[didx], ssem[b], add=True)

        def scatter_wait(t, b):
            didx = ed_v[pl.ds(t * _LANES, _LANES)]
            pltpu.make_async_copy(sout[b], u_sh.at[didx], ssem[b]).wait()

        for k in range(4):
            gather_start(k, k)

        def half(t, i, b, bs, first):
            didx = ed_v[pl.ds(t * _LANES, _LANES)]
            sidx = es_v[pl.ds(t * _LANES, _LANES)]
            sv = plsc.load_gather(sw_v, [sidx])
            dv = plsc.load_gather(sd_v, [didx])
            x = sv + dv
            e = jnp.where(x >= 0, x, 0.2 * x)
            w = jnp.exp(e - cval)
            w_v[pl.ds(_LANES, _LANES)] = w
            plsc.addupdate_scatter(dloc_v, [didx], w)
            gather_wait(t, b)

            if first:
                @pl.when(i > 0)
                def _():
                    scatter_wait(t - 2, bs)
            else:
                scatter_wait(t - 2, bs)

            for r in range(_LANES):
                ws = plsc.load_gather(
                    w_v, [jnp.full((_LANES,), _LANES + r, jnp.int32)])
                for cc in range(D // _LANES):
                    sl = pl.ds(cc * _LANES, _LANES)
                    sout[bs][r, sl] = gin[b][r, sl] * ws

            if isinstance(t, int):
                if t + 4 < nch:
                    gather_start(t + 4, b)
            else:
                @pl.when(t + 4 < nch)
                def _():
                    gather_start(t + 4, b)

            scatter_start(t, bs)

        nq = nch // 4

        def quad(i, carry):
            t0 = 4 * i
            for k in range(4):
                half(t0 + k, i, k, k % 2, k < 2)
            return carry

        lax.fori_loop(0, nq, quad, 0)
        for t in range(4 * nq, nch):
            half(t, 1, t % 4, t % 2, False)
        scatter_wait(nch - 2, (nch - 2) % 2)
        scatter_wait(nch - 1, (nch - 1) % 2)

        pltpu.sync_copy(dloc_v, dall_sh.at[pl.ds(s * NS, NS)])
        plsc.subcore_barrier()

        @pl.when(s < 8)
        def _():
            pltpu.sync_copy(u_sh.at[pl.ds(s * rp8, rp8)], stage_v)
            pltpu.sync_copy(stage_v,
                            u_out.at[pl.ds(c * NS + s * rp8, rp8)])

        @pl.when(s == 8)
        def _():
            pltpu.sync_copy(dall_sh, dall_v)

            def dred(k, carry):
                acc = dall_v[pl.ds(k * _LANES, _LANES)]
                for r in range(1, _NSUB):
                    acc = acc + dall_v[pl.ds(r * NS + k * _LANES, _LANES)]
                dloc_v[pl.ds(k * _LANES, _LANES)] = acc
                return carry

            lax.fori_loop(0, NS // _LANES, dred, 0)
            pltpu.sync_copy(dloc_v, den_out.at[pl.ds(c * NS, NS)])

    return body(hw_p, sw, sd, e_src, e_dst, cvec)


def _tc_word_chain(h_w, Ww, a_src):
    NW, D = h_w.shape
    blk = 640
    grid = NW // blk

    def body(x_ref, w_ref, a_ref, p0_ref, p1_ref, s0_ref, s1_ref, m_ref):
        x = x_ref[...]
        p0 = jnp.dot(x, w_ref[0], preferred_element_type=jnp.float32)
        p0_ref[...] = p0
        s0 = jnp.sum(p0 * a_ref[0:1, :], axis=1, keepdims=True)
        s0_ref[...] = s0
        h1 = _elu(p0)
        p1 = jnp.dot(h1, w_ref[1], preferred_element_type=jnp.float32)
        p1_ref[...] = p1
        s1 = jnp.sum(p1 * a_ref[1:2, :], axis=1, keepdims=True)
        s1_ref[...] = s1
        mx = jnp.concatenate(
            [jnp.max(s0).reshape(1, 1), jnp.max(s1).reshape(1, 1)], axis=1)

        @pl.when(pl.program_id(0) == 0)
        def _():
            m_ref[...] = mx

        @pl.when(pl.program_id(0) > 0)
        def _():
            m_ref[...] = jnp.maximum(m_ref[...], mx)

    return pl.pallas_call(
        body,
        grid=(grid,),
        in_specs=[
            pl.BlockSpec((blk, D), lambda i: (i, 0)),
            pl.BlockSpec((2, D, D), lambda i: (0, 0, 0)),
            pl.BlockSpec((2, D), lambda i: (0, 0)),
        ],
        out_specs=[
            pl.BlockSpec((blk, D), lambda i: (i, 0)),
            pl.BlockSpec((blk, D), lambda i: (i, 0)),
            pl.BlockSpec((blk, 1), lambda i: (i, 0)),
            pl.BlockSpec((blk, 1), lambda i: (i, 0)),
            pl.BlockSpec((1, 2), lambda i: (0, 0)),
        ],
        out_shape=[
            jax.ShapeDtypeStruct((NW, D), jnp.float32),
            jax.ShapeDtypeStruct((NW, D), jnp.float32),
            jax.ShapeDtypeStruct((NW, 1), jnp.float32),
            jax.ShapeDtypeStruct((NW, 1), jnp.float32),
            jax.ShapeDtypeStruct((1, 2), jnp.float32),
        ],
    )(h_w, Ww, a_src)


def _tc_sent_init(h_s, score, Ws0, a_dst0):
    NS, D = h_s.shape
    B, S = score.shape

    def body(hs_ref, sc_ref, w_ref, a_ref, hsp_ref, hsupp_ref, sd_ref, m_ref):
        hs = hs_ref[...]
        hsp = jnp.dot(hs, w_ref[...], preferred_element_type=jnp.float32)
        hsp_ref[...] = hsp
        sd = jnp.sum(hsp * a_ref[...], axis=1, keepdims=True)
        sd_ref[...] = sd
        m_ref[...] = jnp.max(sd).reshape(1, 1)
        rows = []
        for b in range(B):
            sb = sc_ref[b, :].reshape(S, 1)
            rows.append(jnp.sum(sb * hs[b * S:(b + 1) * S, :], axis=0,
                                keepdims=True))
        hsup = jnp.concatenate(rows, axis=0)
        hsupp_ref[...] = jnp.dot(hsup, w_ref[...],
                                 preferred_element_type=jnp.float32)

    return pl.pallas_call(
        body,
        out_shape=[
            jax.ShapeDtypeStruct((NS, D), jnp.float32),
            jax.ShapeDtypeStruct((B, D), jnp.float32),
            jax.ShapeDtypeStruct((NS, 1), jnp.float32),
            jax.ShapeDtypeStruct((1, 1), jnp.float32),
        ],
    )(h_s, score, Ws0, a_dst0)


def _tc_layer0_post(u, den, hs_p, hsup_p, a_s0, a_sup0, Ws1, a_dst1, B, S):
    NS, D = hs_p.shape

    def body(u_ref, d_ref, hsp_ref, hsupp_ref, as_ref, asup_ref,
             w1_ref, ad1_ref, hsp1_ref, hsupp1_ref, sd1_ref, m_ref):
        usum = u_ref[0] + u_ref[1]
        dsum = d_ref[0] + d_ref[1]
        pre = usum / (dsum + 1e-30) + hsp_ref[...]
        hs1 = _elu(pre)
        esup = jnp.sum(hsupp_ref[...] * asup_ref[...], axis=1, keepdims=True)
        ecol = jnp.sum(hs1 * as_ref[...], axis=1, keepdims=True)
        rows = []
        for b in range(B):
            v = ecol[b * S:(b + 1) * S, :] + esup[b, 0]
            lr = _lrelu(v)
            ex = jnp.exp(lr - jnp.max(lr))
            nsb = ex / jnp.sum(ex)
            rows.append(jnp.sum(nsb * hs1[b * S:(b + 1) * S, :], axis=0,
                                keepdims=True))
        hsup1 = _elu(jnp.concatenate(rows, axis=0))
        hsp1 = jnp.dot(hs1, w1_ref[...], preferred_element_type=jnp.float32)
        hsp1_ref[...] = hsp1
        hsupp1_ref[...] = jnp.dot(hsup1, w1_ref[...],
                                  preferred_element_type=jnp.float32)
        sd1 = jnp.sum(hsp1 * ad1_ref[...], axis=1, keepdims=True)
        sd1_ref[...] = sd1
        m_ref[...] = jnp.max(sd1).reshape(1, 1)

    return pl.pallas_call(
        body,
        out_shape=[
            jax.ShapeDtypeStruct((NS, D), jnp.float32),
            jax.ShapeDtypeStruct((B, D), jnp.float32),
            jax.ShapeDtypeStruct((NS, 1), jnp.float32),
            jax.ShapeDtypeStruct((1, 1), jnp.float32),
        ],
    )(u, den, hs_p, hsup_p, a_s0, a_sup0, Ws1, a_dst1)


def _tc_tail(u, den, hs_p, hsup_p, a_s1, a_sup1, W_cls, b_cls, B, S):
    NS, D = hs_p.shape

    def body(u_ref, d_ref, hsp_ref, hsupp_ref, as_ref, asup_ref,
             wc_ref, bc_ref, sh_ref, ns_ref, sup_ref, cls_ref):
        usum = u_ref[0] + u_ref[1]
        dsum = d_ref[0] + d_ref[1]
        pre = usum / (dsum + 1e-30) + hsp_ref[...]
        hs2 = _elu(pre)
        sh_ref[...] = hs2
        esup = jnp.sum(hsupp_ref[...] * asup_ref[...], axis=1, keepdims=True)
        ecol = jnp.sum(hs2 * as_ref[...], axis=1, keepdims=True)
        nt = (((1,), (1,)), ((), ()))
        sup_rows = []
        sps = []
        sq = jnp.zeros((S, S), jnp.float32)
        for b in range(B):
            hb = hs2[b * S:(b + 1) * S, :]
            v = ecol[b * S:(b + 1) * S, :] + esup[b, 0]
            lr = _lrelu(v)
            ex = jnp.exp(lr - jnp.max(lr))
            nsb = ex / jnp.sum(ex)
            ns_ref[b * S:(b + 1) * S, :] = nsb
            sup_rows.append(jnp.sum(nsb * hb, axis=0, keepdims=True))
            wmat = lax.dot_general(nsb, nsb, nt,
                                   preferred_element_type=jnp.float32)
            gram = lax.dot_general(hb, hb, nt,
                                   preferred_element_type=jnp.float32)
            spb = 1.0 / (1.0 + jnp.exp(-(wmat * gram)))
            sps.append(spb)
            sq = sq + spb * spb
        sup_ref[...] = _elu(jnp.concatenate(sup_rows, axis=0))
        nrm = jnp.maximum(jnp.sqrt(sq), 1e-12)
        for b in range(B):
            hb = hs2[b * S:(b + 1) * S, :]
            hagg = jnp.dot(sps[b] / nrm, hb,
                           preferred_element_type=jnp.float32)
            cls_ref[b * S:(b + 1) * S, :] = (
                jnp.dot(hagg, wc_ref[...],
                        preferred_element_type=jnp.float32) + bc_ref[...])

    return pl.pallas_call(
        body,
        out_shape=[
            jax.ShapeDtypeStruct((NS, D), jnp.float32),
            jax.ShapeDtypeStruct((NS, 1), jnp.float32),
            jax.ShapeDtypeStruct((B, D), jnp.float32),
            jax.ShapeDtypeStruct((NS, 2), jnp.float32),
        ],
    )(u, den, hs_p, hsup_p, a_s1, a_sup1, W_cls, b_cls)


def kernel(wids, sentences_wids, edge_src, edge_dst, score, emb,
           Ww, Ws, a_src, a_dst, a_s, a_sup, W_cls, b_cls):
    NW = wids.shape[0]
    NS, L = sentences_wids.shape
    B, S = score.shape
    V, D = emb.shape

    wids = wids.astype(jnp.int32)
    swids = sentences_wids.astype(jnp.int32)
    e_src = edge_src.astype(jnp.int32)
    e_dst = edge_dst.astype(jnp.int32)

    NWP = 640 * _NT
    wids_p = jnp.concatenate(
        [wids, jnp.zeros((NWP - NW,), jnp.int32)]) if NWP > NW else wids
    h_w = _sc_embed_words(emb, wids_p)
    h_s = _sc_embed_sents(emb, swids.reshape(NS * L), NS, L).reshape(NS, D)

    hw_p0, hw_p1, sw0, sw1, msw = _tc_word_chain(h_w, Ww, a_src)
    hs_p0, hsup_p0, sd0, msd0 = _tc_sent_init(h_s, score, Ws[0],
                                              a_dst[0].reshape(1, D))

    c0 = _lrelu(msw[0, 0] + msd0[0, 0])
    u0, den0 = _sc_edges(hw_p0, sw0.reshape(NWP), sd0.reshape(NS),
                         e_src, e_dst, jnp.broadcast_to(c0, (_LANES,)))

    hs_p1, hsup_p1, sd1, msd1 = _tc_layer0_post(
        u0.reshape(_NC, NS, D),
        den0.reshape(_NC, NS, 1), hs_p0, hsup_p0,
        a_s[0].reshape(1, D), a_sup[0].reshape(1, D),
        Ws[1], a_dst[1].reshape(1, D), B, S)

    c1 = _lrelu(msw[0, 1] + msd1[0, 0])
    u1, den1 = _sc_edges(hw_p1, sw1.reshape(NWP), sd1.reshape(NS),
                         e_src, e_dst, jnp.broadcast_to(c1, (_LANES,)))

    s_h, ns_col, super_h, cls_out = _tc_tail(
        u1.reshape(_NC, NS, D),
        den1.reshape(_NC, NS, 1), hs_p1, hsup_p1,
        a_s[1].reshape(1, D), a_sup[1].reshape(1, D),
        W_cls, b_cls.reshape(1, 2), B, S)

    new_score = ns_col.reshape(B, S)
    return (new_score, s_h, super_h, cls_out)


# speedup vs baseline: 23.9933x; 1.0180x over previous
import functools

import jax
import jax.numpy as jnp
from jax import lax
from jax.experimental import pallas as pl
from jax.experimental.pallas import tpu as pltpu
from jax.experimental.pallas import tpu_sc as plsc

_NC = 2
_NSUB = 16
_LANES = 16
_NT = _NC * _NSUB


def _elu(x):
    return jnp.where(x > 0, x, jnp.exp(x) - 1.0)


def _lrelu(x):
    return jnp.where(x >= 0, x, 0.2 * x)


def _sc_embed_words(emb, wids):
    V, D = emb.shape
    NW = wids.shape[0]
    wpt = 640
    gsz = 64
    mesh = plsc.VectorSubcoreMesh(core_axis_name="c", subcore_axis_name="s")

    @functools.partial(
        pl.kernel,
        out_type=jax.ShapeDtypeStruct((NW, D), jnp.float32),
        mesh=mesh,
        scratch_types=[
            pltpu.VMEM((wpt,), jnp.int32),
            pltpu.VMEM((gsz, D), jnp.float32),
            pltpu.VMEM((gsz, D), jnp.float32),
            pltpu.SemaphoreType.DMA,
            pltpu.SemaphoreType.DMA,
            pltpu.SemaphoreType.DMA,
            pltpu.SemaphoreType.DMA,
        ],
    )
    def body(emb_h, wids_h, hw_out, widx_v, r0_v, r1_v, g0, g1, w0, w1):
        c = lax.axis_index("c")
        s = lax.axis_index("s")
        wid = s * _NC + c
        base = wid * wpt
        n_ch = wpt // gsz
        rows = (r0_v, r1_v)
        gsem = (g0, g1)
        wsem = (w0, w1)

        pltpu.sync_copy(wids_h.at[pl.ds(base, wpt)], widx_v)

        def g_start(k, b):
            pltpu.async_copy(emb_h.at[widx_v.at[pl.ds(k * gsz, gsz)]],
                             rows[b], gsem[b])

        def g_wait(k, b):
            pltpu.make_async_copy(emb_h.at[widx_v.at[pl.ds(k * gsz, gsz)]],
                                  rows[b], gsem[b]).wait()

        def w_start(k, b):
            pltpu.async_copy(rows[b], hw_out.at[pl.ds(base + k * gsz, gsz)],
                             wsem[b])

        def w_wait(k, b):
            pltpu.make_async_copy(rows[b],
                                  hw_out.at[pl.ds(base + k * gsz, gsz)],
                                  wsem[b]).wait()

        g_start(0, 0)
        g_start(1, 1)

        def step(k, carry):
            for b in range(2):
                t = 2 * k + b
                g_wait(t, b)
                w_start(t, b)

                @pl.when(t + 2 < n_ch)
                def _():
                    w_wait(t, b)
                    g_start(t + 2, b)
            return carry

        lax.fori_loop(0, n_ch // 2, step, 0)
        w_wait(n_ch - 2, 0)
        w_wait(n_ch - 1, 1)

    return body(emb, wids)


def _sc_embed_sents(emb, swids_flat, NS, L):
    V, D = emb.shape
    sent_per = NS // _NT
    nst = sent_per // 2
    mesh = plsc.VectorSubcoreMesh(core_axis_name="c", subcore_axis_name="s")

    @functools.partial(
        pl.kernel,
        out_type=jax.ShapeDtypeStruct((NS * D,), jnp.float32),
        mesh=mesh,
        scratch_types=[
            pltpu.VMEM((sent_per * L,), jnp.int32),
            pltpu.VMEM((2 * L, D), jnp.float32),
            pltpu.VMEM((2 * L, D), jnp.float32),
            pltpu.VMEM((sent_per * D,), jnp.float32),
            pltpu.SemaphoreType.DMA,
            pltpu.SemaphoreType.DMA,
        ],
    )
    def body(emb_h, swids_h, hs_out, sidx_v, s0_v, s1_v, pool_v, g0, g1):
        c = lax.axis_index("c")
        s = lax.axis_index("s")
        wid = s * _NC + c
        srows = (s0_v, s1_v)
        gsem = (g0, g1)

        pltpu.sync_copy(swids_h.at[pl.ds(wid * sent_per * L, sent_per * L)],
                        sidx_v)

        def g_start(j, b):
            pltpu.async_copy(emb_h.at[sidx_v.at[pl.ds(j * 2 * L, 2 * L)]],
                             srows[b], gsem[b])

        def g_wait(j, b):
            pltpu.make_async_copy(
                emb_h.at[sidx_v.at[pl.ds(j * 2 * L, 2 * L)]],
                srows[b], gsem[b]).wait()

        g_start(0, 0)
        g_start(1, 1)

        def pool2(j, b):
            g_wait(j, b)
            for half in range(2):
                for cc in range(D // _LANES):
                    acc = srows[b][half * L, pl.ds(cc * _LANES, _LANES)]
                    for r in range(1, L):
                        acc = acc + srows[b][half * L + r,
                                             pl.ds(cc * _LANES, _LANES)]
                    pool_v[pl.ds((2 * j + half) * D + cc * _LANES,
                                 _LANES)] = acc * (1.0 / L)

            if isinstance(j, int):
                if j + 2 < nst:
                    g_start(j + 2, b)
            else:
                @pl.when(j + 2 < nst)
                def _():
                    g_start(j + 2, b)

        def step(k, carry):
            pool2(2 * k, 0)
            pool2(2 * k + 1, 1)
            return carry

        lax.fori_loop(0, nst // 2, step, 0)
        for j in range(2 * (nst // 2), nst):
            pool2(j, j % 2)
        pltpu.sync_copy(pool_v,
                        hs_out.at[pl.ds(wid * sent_per * D, sent_per * D)])

    return body(emb, swids_flat)


def _sc_edges(hw_p, sw, sd, e_src, e_dst, cvec):
    NW, D = hw_p.shape
    NS = sd.shape[0]
    E = e_src.shape[0]
    ept = E // _NT
    nch = ept // _LANES
    rp8 = NS // 8
    mesh = plsc.VectorSubcoreMesh(core_axis_name="c", subcore_axis_name="s")

    @functools.partial(
        pl.kernel,
        out_type=[
            jax.ShapeDtypeStruct((_NC * NS, D), jnp.float32),
            jax.ShapeDtypeStruct((_NC * NS,), jnp.float32),
        ],
        mesh=mesh,
        scratch_types=[
            pltpu.VMEM((NW,), jnp.float32),
            pltpu.VMEM((NS,), jnp.float32),
            pltpu.VMEM((_LANES,), jnp.float32),
            pltpu.VMEM((ept,), jnp.int32),
            pltpu.VMEM((ept,), jnp.int32),
            pltpu.VMEM((2 * _LANES,), jnp.float32),
            pltpu.VMEM((_LANES, D), jnp.float32),
            pltpu.VMEM((_LANES, D), jnp.float32),
            pltpu.VMEM((_LANES, D), jnp.float32),
            pltpu.VMEM((_LANES, D), jnp.float32),
            pltpu.VMEM((_LANES, D), jnp.float32),
            pltpu.VMEM((_LANES, D), jnp.float32),
            pltpu.VMEM((rp8, D), jnp.float32),
            pltpu.VMEM((NS,), jnp.float32),
            pltpu.VMEM((_NSUB * NS,), jnp.float32),
            pltpu.VMEM_SHARED((NS, D), jnp.float32),
            pltpu.VMEM_SHARED((_NSUB * NS,), jnp.float32),
            pltpu.SemaphoreType.DMA,
            pltpu.SemaphoreType.DMA,
            pltpu.SemaphoreType.DMA,
            pltpu.SemaphoreType.DMA,
            pltpu.SemaphoreType.DMA,
            pltpu.SemaphoreType.DMA,
        ],
        compiler_params=pltpu.CompilerParams(needs_layout_passes=False),
    )
    def body(hwp_h, sw_h, sd_h, es_h, ed_h, c_h,
             u_out, den_out,
             sw_v, sd_v, c_v, es_v, ed_v, w_v,
             gin0_v, gin1_v, gin2_v, gin3_v, sout0_v, sout1_v,
             stage_v, dloc_v, dall_v, u_sh, dall_sh,
             gsem0, gsem1, gsem2, gsem3, ssem0, ssem1):
        gin = (gin0_v, gin1_v, gin2_v, gin3_v)
        sout = (sout0_v, sout1_v)
        gsem = (gsem0, gsem1, gsem2, gsem3)
        ssem = (ssem0, ssem1)
        c = lax.axis_index("c")
        s = lax.axis_index("s")
        wid = s * _NC + c

        zv = jnp.zeros((_LANES,), jnp.float32)

        @pl.when(s < 8)
        def _():
            def zrow(i, carry):
                for cc in range(D // _LANES):
                    stage_v[i, pl.ds(cc * _LANES, _LANES)] = zv
                return carry

            lax.fori_loop(0, rp8, zrow, 0)
            pltpu.sync_copy(stage_v, u_sh.at[pl.ds(s * rp8, rp8)])

        def zden(i, carry):
            dloc_v[pl.ds(i * _LANES, _LANES)] = zv
            return carry

        lax.fori_loop(0, NS // _LANES, zden, 0)

        pltpu.async_copy(sw_h, sw_v, gsem0)
        pltpu.async_copy(sd_h, sd_v, gsem0)
        pltpu.async_copy(c_h, c_v, gsem0)
        pltpu.async_copy(es_h.at[pl.ds(wid * ept, ept)], es_v, gsem0)
        pltpu.async_copy(ed_h.at[pl.ds(wid * ept, ept)], ed_v, gsem0)
        pltpu.make_async_copy(sw_h, sw_v, gsem0).wait()
        pltpu.make_async_copy(sd_h, sd_v, gsem0).wait()
        pltpu.make_async_copy(c_h, c_v, gsem0).wait()
        pltpu.make_async_copy(es_h.at[pl.ds(wid * ept, ept)], es_v,
                              gsem0).wait()
        pltpu.make_async_copy(ed_h.at[pl.ds(wid * ept, ept)], ed_v,
                              gsem0).wait()
        plsc.subcore_barrier()

        cval = c_v[...]

        def gather_start(t, b):
            sidx = es_v[pl.ds(t * _LANES, _LANES)]
            pltpu.async_copy(hwp_h.at[sidx], gin[b], gsem[b])

        def gather_wait(t, b):
            sidx = es_v[pl.ds(t * _LANES, _LANES)]
            pltpu.make_async_copy(hwp_h.at[sidx], gin[b], gsem[b]).wait()

        def scatter_start(t, b):
            didx = ed_v[pl.ds(t * _LANES, _LANES)]
            pltpu.async_copy(sout[b], u_sh.at[didx], ssem[b], add=True)

        def scatter_wait(t, b):
            didx = ed_v[pl.ds(t * _LANES, _LANES)]
            pltpu.make_async_copy(sout[b], u_sh.at[didx], ssem[b]).wait()

        for k in range(4):
            gather_start(k, k)

        def half(t, i, b, bs, first):
            didx = ed_v[pl.ds(t * _LANES, _LANES)]
            sidx = es_v[pl.ds(t * _LANES, _LANES)]
            sv = plsc.load_gather(sw_v, [sidx])
            dv = plsc.load_gather(sd_v, [didx])
            x = sv + dv
            e = jnp.where(x >= 0, x, 0.2 * x)
            w = jnp.exp(e - cval)
            w_v[pl.ds(_LANES, _LANES)] = w
            plsc.addupdate_scatter(dloc_v, [didx], w)
            gather_wait(t, b)

            if first:
                @pl.when(i > 0)
                def _():
                    scatter_wait(t - 2, bs)
            else:
                scatter_wait(t - 2, bs)

            for r in range(_LANES):
                ws = plsc.load_gather(
                    w_v, [jnp.full((_LANES,), _LANES + r, jnp.int32)])
                for cc in range(D // _LANES):
                    sl = pl.ds(cc * _LANES, _LANES)
                    sout[bs][r, sl] = gin[b][r, sl] * ws

            if isinstance(t, int):
                if t + 4 < nch:
                    gather_start(t + 4, b)
            else:
                @pl.when(t + 4 < nch)
                def _():
                    gather_start(t + 4, b)

            scatter_start(t, bs)

        nq = nch // 4

        def quad(i, carry):
            t0 = 4 * i
            for k in range(4):
                half(t0 + k, i, k, k % 2, k < 2)
            return carry

        lax.fori_loop(0, nq, quad, 0)
        for t in range(4 * nq, nch):
            half(t, 1, t % 4, t % 2, False)
        scatter_wait(nch - 2, (nch - 2) % 2)
        scatter_wait(nch - 1, (nch - 1) % 2)

        pltpu.sync_copy(dloc_v, dall_sh.at[pl.ds(s * NS, NS)])
        plsc.subcore_barrier()

        @pl.when(s < 8)
        def _():
            pltpu.sync_copy(u_sh.at[pl.ds(s * rp8, rp8)], stage_v)
            pltpu.sync_copy(stage_v,
                            u_out.at[pl.ds(c * NS + s * rp8, rp8)])

        @pl.when(s == 8)
        def _():
            pltpu.sync_copy(dall_sh, dall_v)

            def dred(k, carry):
                acc = dall_v[pl.ds(k * _LANES, _LANES)]
                for r in range(1, _NSUB):
                    acc = acc + dall_v[pl.ds(r * NS + k * _LANES, _LANES)]
                dloc_v[pl.ds(k * _LANES, _LANES)] = acc
                return carry

            lax.fori_loop(0, NS // _LANES, dred, 0)
            pltpu.sync_copy(dloc_v, den_out.at[pl.ds(c * NS, NS)])

    return body(hw_p, sw, sd, e_src, e_dst, cvec)


def _tc_word_chain(h_w, Ww, a_src):
    NW, D = h_w.shape
    blk = 640
    grid = NW // blk

    def body(x_ref, w_ref, a_ref, p0_ref, p1_ref, s0_ref, s1_ref, m_ref):
        x = x_ref[...]
        p0 = jnp.dot(x, w_ref[0], preferred_element_type=jnp.float32)
        p0_ref[...] = p0
        s0 = jnp.sum(p0 * a_ref[0:1, :], axis=1, keepdims=True)
        s0_ref[...] = s0
        h1 = _elu(p0)
        p1 = jnp.dot(h1, w_ref[1], preferred_element_type=jnp.float32)
        p1_ref[...] = p1
        s1 = jnp.sum(p1 * a_ref[1:2, :], axis=1, keepdims=True)
        s1_ref[...] = s1
        mx = jnp.concatenate(
            [jnp.max(s0).reshape(1, 1), jnp.max(s1).reshape(1, 1)], axis=1)

        @pl.when(pl.program_id(0) == 0)
        def _():
            m_ref[...] = mx

        @pl.when(pl.program_id(0) > 0)
        def _():
            m_ref[...] = jnp.maximum(m_ref[...], mx)

    return pl.pallas_call(
        body,
        grid=(grid,),
        in_specs=[
            pl.BlockSpec((blk, D), lambda i: (i, 0)),
            pl.BlockSpec((2, D, D), lambda i: (0, 0, 0)),
            pl.BlockSpec((2, D), lambda i: (0, 0)),
        ],
        out_specs=[
            pl.BlockSpec((blk, D), lambda i: (i, 0)),
            pl.BlockSpec((blk, D), lambda i: (i, 0)),
            pl.BlockSpec((blk, 1), lambda i: (i, 0)),
            pl.BlockSpec((blk, 1), lambda i: (i, 0)),
            pl.BlockSpec((1, 2), lambda i: (0, 0)),
        ],
        out_shape=[
            jax.ShapeDtypeStruct((NW, D), jnp.float32),
            jax.ShapeDtypeStruct((NW, D), jnp.float32),
            jax.ShapeDtypeStruct((NW, 1), jnp.float32),
            jax.ShapeDtypeStruct((NW, 1), jnp.float32),
            jax.ShapeDtypeStruct((1, 2), jnp.float32),
        ],
    )(h_w, Ww, a_src)


def _tc_sent_init(h_s, score, Ws0, a_dst0):
    NS, D = h_s.shape
    B, S = score.shape

    def body(hs_ref, sc_ref, w_ref, a_ref, hsp_ref, hsupp_ref, sd_ref, m_ref):
        hs = hs_ref[...]
        hsp = jnp.dot(hs, w_ref[...], preferred_element_type=jnp.float32)
        hsp_ref[...] = hsp
        sd = jnp.sum(hsp * a_ref[...], axis=1, keepdims=True)
        sd_ref[...] = sd
        m_ref[...] = jnp.max(sd).reshape(1, 1)
        rows = []
        for b in range(B):
            sb = sc_ref[b, :].reshape(S, 1)
            rows.append(jnp.sum(sb * hs[b * S:(b + 1) * S, :], axis=0,
                                keepdims=True))
        hsup = jnp.concatenate(rows, axis=0)
        hsupp_ref[...] = jnp.dot(hsup, w_ref[...],
                                 preferred_element_type=jnp.float32)

    return pl.pallas_call(
        body,
        out_shape=[
            jax.ShapeDtypeStruct((NS, D), jnp.float32),
            jax.ShapeDtypeStruct((B, D), jnp.float32),
            jax.ShapeDtypeStruct((NS, 1), jnp.float32),
            jax.ShapeDtypeStruct((1, 1), jnp.float32),
        ],
    )(h_s, score, Ws0, a_dst0)


def _tc_layer0_post(u, den, hs_p, hsup_p, a_s0, a_sup0, Ws1, a_dst1, B, S):
    NS, D = hs_p.shape

    def body(u_ref, d_ref, hsp_ref, hsupp_ref, as_ref, asup_ref,
             w1_ref, ad1_ref, hsp1_ref, hsupp1_ref, sd1_ref, m_ref):
        usum = u_ref[0] + u_ref[1]
        dsum = d_ref[0] + d_ref[1]
        pre = usum / (dsum + 1e-30) + hsp_ref[...]
        hs1 = _elu(pre)
        esup = jnp.sum(hsupp_ref[...] * asup_ref[...], axis=1, keepdims=True)
        ecol = jnp.sum(hs1 * as_ref[...], axis=1, keepdims=True)
        rows = []
        for b in range(B):
            v = ecol[b * S:(b + 1) * S, :] + esup[b, 0]
            lr = _lrelu(v)
            ex = jnp.exp(lr - jnp.max(lr))
            nsb = ex / jnp.sum(ex)
            rows.append(jnp.sum(nsb * hs1[b * S:(b + 1) * S, :], axis=0,
                                keepdims=True))
        hsup1 = _elu(jnp.concatenate(rows, axis=0))
        hsp1 = jnp.dot(hs1, w1_ref[...], preferred_element_type=jnp.float32)
        hsp1_ref[...] = hsp1
        hsupp1_ref[...] = jnp.dot(hsup1, w1_ref[...],
                                  preferred_element_type=jnp.float32)
        sd1 = jnp.sum(hsp1 * ad1_ref[...], axis=1, keepdims=True)
        sd1_ref[...] = sd1
        m_ref[...] = jnp.max(sd1).reshape(1, 1)

    return pl.pallas_call(
        body,
        out_shape=[
            jax.ShapeDtypeStruct((NS, D), jnp.float32),
            jax.ShapeDtypeStruct((B, D), jnp.float32),
            jax.ShapeDtypeStruct((NS, 1), jnp.float32),
            jax.ShapeDtypeStruct((1, 1), jnp.float32),
        ],
    )(u, den, hs_p, hsup_p, a_s0, a_sup0, Ws1, a_dst1)


def _tc_tail(u, den, hs_p, hsup_p, a_s1, a_sup1, W_cls, b_cls, B, S):
    NS, D = hs_p.shape

    def body(u_ref, d_ref, hsp_ref, hsupp_ref, as_ref, asup_ref,
             wc_ref, bc_ref, sh_ref, ns_ref, sup_ref, cls_ref):
        usum = u_ref[0] + u_ref[1]
        dsum = d_ref[0] + d_ref[1]
        pre = usum / (dsum + 1e-30) + hsp_ref[...]
        hs2 = _elu(pre)
        sh_ref[...] = hs2
        esup = jnp.sum(hsupp_ref[...] * asup_ref[...], axis=1, keepdims=True)
        ecol = jnp.sum(hs2 * as_ref[...], axis=1, keepdims=True)
        nt = (((1,), (1,)), ((), ()))
        sup_rows = []
        sps = []
        sq = jnp.zeros((S, S), jnp.float32)
        for b in range(B):
            hb = hs2[b * S:(b + 1) * S, :]
            v = ecol[b * S:(b + 1) * S, :] + esup[b, 0]
            lr = _lrelu(v)
            ex = jnp.exp(lr - jnp.max(lr))
            nsb = ex / jnp.sum(ex)
            ns_ref[b * S:(b + 1) * S, :] = nsb
            sup_rows.append(jnp.sum(nsb * hb, axis=0, keepdims=True))
            wmat = lax.dot_general(nsb, nsb, nt,
                                   preferred_element_type=jnp.float32)
            gram = lax.dot_general(hb, hb, nt,
                                   preferred_element_type=jnp.float32)
            spb = 1.0 / (1.0 + jnp.exp(-(wmat * gram)))
            sps.append(spb)
            sq = sq + spb * spb
        sup_ref[...] = _elu(jnp.concatenate(sup_rows, axis=0))
        nrm = jnp.maximum(jnp.sqrt(sq), 1e-12)
        for b in range(B):
            hb = hs2[b * S:(b + 1) * S, :]
            hagg = jnp.dot(sps[b] / nrm, hb,
                           preferred_element_type=jnp.float32)
            cls_ref[b * S:(b + 1) * S, :] = (
                jnp.dot(hagg, wc_ref[...],
                        preferred_element_type=jnp.float32) + bc_ref[...])

    return pl.pallas_call(
        body,
        out_shape=[
            jax.ShapeDtypeStruct((NS, D), jnp.float32),
            jax.ShapeDtypeStruct((NS, 1), jnp.float32),
            jax.ShapeDtypeStruct((B, D), jnp.float32),
            jax.ShapeDtypeStruct((NS, 2), jnp.float32),
        ],
    )(u, den, hs_p, hsup_p, a_s1, a_sup1, W_cls, b_cls)


def kernel(wids, sentences_wids, edge_src, edge_dst, score, emb,
           Ww, Ws, a_src, a_dst, a_s, a_sup, W_cls, b_cls):
    NW = wids.shape[0]
    NS, L = sentences_wids.shape
    B, S = score.shape
    V, D = emb.shape

    wids = wids.astype(jnp.int32)
    swids = sentences_wids.astype(jnp.int32)
    e_src = edge_src.astype(jnp.int32)
    e_dst = edge_dst.astype(jnp.int32)

    NWP = 640 * _NT
    wids_p = jnp.concatenate(
        [wids, jnp.zeros((NWP - NW,), jnp.int32)]) if NWP > NW else wids
    h_w = _sc_embed_words(emb, wids_p)
    h_s = _sc_embed_sents(emb, swids.reshape(NS * L), NS, L).reshape(NS, D)

    hw_p0, hw_p1, sw0, sw1, msw = _tc_word_chain(h_w, Ww, a_src)
    hs_p0, hsup_p0, sd0, msd0 = _tc_sent_init(h_s, score, Ws[0],
                                              a_dst[0].reshape(1, D))

    c0 = _lrelu(msw[0, 0] + msd0[0, 0])
    u0, den0 = _sc_edges(hw_p0, sw0.reshape(NWP), sd0.reshape(NS),
                         e_src, e_dst, jnp.broadcast_to(c0, (_LANES,)))

    hs_p1, hsup_p1, sd1, msd1 = _tc_layer0_post(
        u0.reshape(_NC, NS, D),
        den0.reshape(_NC, NS, 1), hs_p0, hsup_p0,
        a_s[0].reshape(1, D), a_sup[0].reshape(1, D),
        Ws[1], a_dst[1].reshape(1, D), B, S)

    c1 = _lrelu(msw[0, 1] + msd1[0, 0])
    u1, den1 = _sc_edges(hw_p1, sw1.reshape(NWP), sd1.reshape(NS),
                         e_src, e_dst, jnp.broadcast_to(c1, (_LANES,)))

    s_h, ns_col, super_h, cls_out = _tc_tail(
        u1.reshape(_NC, NS, D),
        den1.reshape(_NC, NS, 1), hs_p1, hsup_p1,
        a_s[1].reshape(1, D), a_sup[1].reshape(1, D),
        W_cls, b_cls.reshape(1, 2), B, S)

    new_score = ns_col.reshape(B, S)
    return (new_score, s_h, super_h, cls_out)
